# Initial kernel scaffold; baseline (speedup 1.0000x reference)
import jax
import jax.numpy as jnp
from jax.experimental import pallas as pl


def kernel(edge_index, length):
    raise NotImplementedError("write your pallas kernel here")


# speedup vs baseline: 1.2539x; 1.2539x over previous
import functools

import jax
import jax.numpy as jnp
from jax import lax
from jax.experimental import pallas as pl
from jax.experimental.pallas import tpu as pltpu
from jax.experimental.pallas import tpu_sc as plsc

E = 6_400_000
N = 100_000
NC = 2
NS = 16
NW = NC * NS
EW = E // NW
NB = 3_136
NPAD = NB * NW
CH = 4_000
NCH = EW // CH
CB = 784
L = 16

_mesh = plsc.VectorSubcoreMesh(core_axis_name="c", subcore_axis_name="s")
_params = pltpu.CompilerParams(
    needs_layout_passes=False, use_tc_tiling_on_sc=False)

i32 = jnp.int32


def _wid():
  return lax.axis_index("s") * NC + lax.axis_index("c")


@functools.partial(
    pl.kernel,
    out_type=(
        jax.ShapeDtypeStruct((NW, NPAD), i32),
        jax.ShapeDtypeStruct((NW, NW), i32),
    ),
    mesh=_mesh,
    compiler_params=_params,
    scratch_types=[
        pltpu.VMEM((NPAD,), i32),
        pltpu.VMEM((CH,), i32),
        pltpu.VMEM((NW,), i32),
    ],
)
def _hist_kernel(src_hbm, hist_hbm, psum_hbm, hist_v, src_v, psum_v):
  wid = _wid()

  @plsc.parallel_loop(0, NPAD // L, unroll=8)
  def _(i):
    hist_v[pl.ds(i * L, L)] = jnp.zeros((L,), i32)

  def chunk_body(ci, _):
    base = wid * EW + ci * CH
    pltpu.sync_copy(src_hbm.at[pl.ds(base, CH)], src_v)

    def vec_body(i, _):
      v = src_v[pl.ds(i * L, L)]
      cnt, last = plsc.scan_count(v)
      plsc.addupdate_scatter(hist_v, [v], cnt, mask=last)
      return 0

    lax.fori_loop(0, CH // L, vec_body, 0)
    return 0

  lax.fori_loop(0, NCH, chunk_body, 0)

  lane0 = lax.iota(i32, L) == 0
  for r in range(NW):
    def sum_body(j, acc):
      return acc + hist_v[pl.ds(r * NB + j * L, L)]

    acc = lax.fori_loop(0, NB // L, sum_body, jnp.zeros((L,), i32))
    total = jnp.sum(acc)
    plsc.store_scatter(
        psum_v, [jnp.full((L,), r, i32)], jnp.full((L,), total, i32),
        mask=lane0)

  pltpu.sync_copy(hist_v, hist_hbm.at[wid])
  pltpu.sync_copy(psum_v, psum_hbm.at[wid])


@functools.partial(
    pl.kernel,
    out_type=(
        jax.ShapeDtypeStruct((NPAD,), i32),
        jax.ShapeDtypeStruct((NW, NPAD), i32),
    ),
    mesh=_mesh,
    compiler_params=_params,
    scratch_types=[
        pltpu.VMEM((NW, NW), i32),
        pltpu.VMEM((NW, CB), i32),
        pltpu.VMEM((NW, CB), i32),
        pltpu.VMEM((CB,), i32),
    ],
)
def _offsets_kernel(hist_hbm, psum_hbm, splits_hbm, woff_hbm, psum_v, hcol_v,
                    woff_v, spl_v):
  wid = _wid()
  pltpu.sync_copy(psum_hbm, psum_v)

  acc0 = jnp.zeros((L,), i32)
  acc1 = jnp.zeros((L,), i32)
  for w2 in range(NW):
    acc0 = acc0 + psum_v[w2, pl.ds(0, L)]
    acc1 = acc1 + psum_v[w2, pl.ds(L, L)]
  iota = lax.iota(i32, L)
  zero = jnp.zeros((L,), i32)
  base = jnp.sum(jnp.where(iota < wid, acc0, zero)) + jnp.sum(
      jnp.where(iota + L < wid, acc1, zero))

  def sub_chunk(k, carry):
    off = wid * NB + k * CB
    pltpu.sync_copy(hist_hbm.at[:, pl.ds(off, CB)], hcol_v)

    def vec_body(i, c):
      tot = jnp.zeros((L,), i32)
      for w2 in range(NW):
        tot = tot + hcol_v[w2, pl.ds(i * L, L)]
      incl = plsc.cumsum(tot) + jnp.full((L,), c, i32)
      spl_v[pl.ds(i * L, L)] = incl
      run = incl - tot
      for w2 in range(NW):
        woff_v[w2, pl.ds(i * L, L)] = run
        run = run + hcol_v[w2, pl.ds(i * L, L)]
      return c + jnp.sum(tot)

    carry = lax.fori_loop(0, CB // L, vec_body, carry)
    pltpu.sync_copy(woff_v, woff_hbm.at[:, pl.ds(off, CB)])
    pltpu.sync_copy(spl_v, splits_hbm.at[pl.ds(off, CB)])
    return carry

  lax.fori_loop(0, NB // CB, sub_chunk, base)


@functools.partial(
    pl.kernel,
    out_type=jax.ShapeDtypeStruct((E,), i32),
    mesh=_mesh,
    compiler_params=_params,
    scratch_types=[
        pltpu.VMEM((NPAD,), i32),
        pltpu.VMEM((CH,), i32),
        pltpu.VMEM((CH,), i32),
        pltpu.VMEM((CH,), i32),
        pltpu.SemaphoreType.DMA,
    ],
)
def _scatter_kernel(src_hbm, tgt_hbm, woff_hbm, out_hbm, woff_v, src_v, tgt_v,
                    pos_v, sem):
  wid = _wid()
  pltpu.sync_copy(woff_hbm.at[wid], woff_v)

  def chunk_body(ci, _):
    base = wid * EW + ci * CH
    pltpu.sync_copy(src_hbm.at[pl.ds(base, CH)], src_v)
    pltpu.sync_copy(tgt_hbm.at[pl.ds(base, CH)], tgt_v)

    def vec_body(i, _):
      v = src_v[pl.ds(i * L, L)]
      cnt, last = plsc.scan_count(v)
      b = plsc.load_gather(woff_v, [v])
      pos_v[pl.ds(i * L, L)] = b + cnt - 1
      plsc.store_scatter(woff_v, [v], b + cnt, mask=last)
      return 0

    lax.fori_loop(0, CH // L, vec_body, 0)
    pltpu.async_copy(tgt_v, out_hbm.at[pos_v], sem).wait()
    return 0

  lax.fori_loop(0, NCH, chunk_body, 0)


@jax.jit
def _crs_neighbor(edge_index):
  src = edge_index[0].astype(i32)
  tgt = edge_index[1].astype(i32)
  hist, psum = _hist_kernel(src)
  splits_body, woff = _offsets_kernel(hist, psum)
  nbr = _scatter_kernel(src, tgt, woff)
  splits = jnp.concatenate(
      [jnp.zeros((1,), i32), splits_body[:N]]).astype(jnp.int64)
  return nbr.astype(jnp.int64), splits


def kernel(edge_index, length):
  del length
  return _crs_neighbor(edge_index)


# speedup vs baseline: 1.2541x; 1.0001x over previous
import functools

import jax
import jax.numpy as jnp
from jax import lax
from jax.experimental import pallas as pl
from jax.experimental.pallas import tpu as pltpu
from jax.experimental.pallas import tpu_sc as plsc

E = 6_400_000
N = 100_000
NC = 2
NS = 16
NW = NC * NS
EW = E // NW
NB = 3_136
NPAD = NB * NW
CH = 4_000
NCH = EW // CH
SCH = 2_000
KR = 4
NG = EW // (SCH * KR)
CB = 784
L = 16

_mesh = plsc.VectorSubcoreMesh(core_axis_name="c", subcore_axis_name="s")
_params = pltpu.CompilerParams(
    needs_layout_passes=False, use_tc_tiling_on_sc=False)

i32 = jnp.int32


def _wid():
  return lax.axis_index("s") * NC + lax.axis_index("c")


@functools.partial(
    pl.kernel,
    out_type=(
        jax.ShapeDtypeStruct((NW, NPAD), i32),
        jax.ShapeDtypeStruct((NW, NW), i32),
    ),
    mesh=_mesh,
    compiler_params=_params,
    scratch_types=[
        pltpu.VMEM((NPAD,), i32),
        pltpu.VMEM((CH,), i32),
        pltpu.VMEM((NW,), i32),
    ],
)
def _hist_kernel(src_hbm, hist_hbm, psum_hbm, hist_v, src_v, psum_v):
  wid = _wid()

  @plsc.parallel_loop(0, NPAD // L, unroll=8)
  def _(i):
    hist_v[pl.ds(i * L, L)] = jnp.zeros((L,), i32)

  def chunk_body(ci, _):
    base = wid * EW + ci * CH
    pltpu.sync_copy(src_hbm.at[pl.ds(base, CH)], src_v)

    def vec_body(i, _):
      v = src_v[pl.ds(i * L, L)]
      cnt, last = plsc.scan_count(v)
      plsc.addupdate_scatter(hist_v, [v], cnt, mask=last)
      return 0

    lax.fori_loop(0, CH // L, vec_body, 0)
    return 0

  lax.fori_loop(0, NCH, chunk_body, 0)

  lane0 = lax.iota(i32, L) == 0
  for r in range(NW):
    def sum_body(j, acc):
      return acc + hist_v[pl.ds(r * NB + j * L, L)]

    acc = lax.fori_loop(0, NB // L, sum_body, jnp.zeros((L,), i32))
    total = jnp.sum(acc)
    plsc.store_scatter(
        psum_v, [jnp.full((L,), r, i32)], jnp.full((L,), total, i32),
        mask=lane0)

  pltpu.sync_copy(hist_v, hist_hbm.at[wid])
  pltpu.sync_copy(psum_v, psum_hbm.at[wid])


@functools.partial(
    pl.kernel,
    out_type=(
        jax.ShapeDtypeStruct((NPAD,), i32),
        jax.ShapeDtypeStruct((NW, NPAD), i32),
    ),
    mesh=_mesh,
    compiler_params=_params,
    scratch_types=[
        pltpu.VMEM((NW, NW), i32),
        pltpu.VMEM((NW, CB), i32),
        pltpu.VMEM((NW, CB), i32),
        pltpu.VMEM((CB,), i32),
    ],
)
def _offsets_kernel(hist_hbm, psum_hbm, splits_hbm, woff_hbm, psum_v, hcol_v,
                    woff_v, spl_v):
  wid = _wid()
  pltpu.sync_copy(psum_hbm, psum_v)

  acc0 = jnp.zeros((L,), i32)
  acc1 = jnp.zeros((L,), i32)
  for w2 in range(NW):
    acc0 = acc0 + psum_v[w2, pl.ds(0, L)]
    acc1 = acc1 + psum_v[w2, pl.ds(L, L)]
  iota = lax.iota(i32, L)
  zero = jnp.zeros((L,), i32)
  base = jnp.sum(jnp.where(iota < wid, acc0, zero)) + jnp.sum(
      jnp.where(iota + L < wid, acc1, zero))

  def sub_chunk(k, carry):
    off = wid * NB + k * CB
    pltpu.sync_copy(hist_hbm.at[:, pl.ds(off, CB)], hcol_v)

    def vec_body(i, c):
      tot = jnp.zeros((L,), i32)
      for w2 in range(NW):
        tot = tot + hcol_v[w2, pl.ds(i * L, L)]
      incl = plsc.cumsum(tot) + jnp.full((L,), c, i32)
      spl_v[pl.ds(i * L, L)] = incl
      run = incl - tot
      for w2 in range(NW):
        woff_v[w2, pl.ds(i * L, L)] = run
        run = run + hcol_v[w2, pl.ds(i * L, L)]
      return c + jnp.sum(tot)

    carry = lax.fori_loop(0, CB // L, vec_body, carry)
    pltpu.sync_copy(woff_v, woff_hbm.at[:, pl.ds(off, CB)])
    pltpu.sync_copy(spl_v, splits_hbm.at[pl.ds(off, CB)])
    return carry

  lax.fori_loop(0, NB // CB, sub_chunk, base)


@functools.partial(
    pl.kernel,
    out_type=jax.ShapeDtypeStruct((E,), i32),
    mesh=_mesh,
    compiler_params=_params,
    scratch_types=[
        pltpu.VMEM((NPAD,), i32),
        pltpu.VMEM((KR, SCH), i32),
        pltpu.VMEM((KR, SCH), i32),
        pltpu.VMEM((KR, SCH), i32),
        [pltpu.SemaphoreType.DMA] * KR,
    ],
)
def _scatter_kernel(src_hbm, tgt_hbm, woff_hbm, out_hbm, woff_v, src_v, tgt_v,
                    pos_v, sems):
  wid = _wid()
  pltpu.sync_copy(woff_hbm.at[wid], woff_v)

  def group_body(g, _):
    for k in range(KR):
      ci = g * KR + k

      @pl.when(g > 0)
      def _():
        pltpu.make_async_copy(
            tgt_v.at[k], out_hbm.at[pos_v.at[k]], sems[k]).wait()

      base = wid * EW + ci * SCH
      pltpu.sync_copy(src_hbm.at[pl.ds(base, SCH)], src_v.at[k])
      pltpu.sync_copy(tgt_hbm.at[pl.ds(base, SCH)], tgt_v.at[k])

      def vec_body(i, _):
        v = src_v[k, pl.ds(i * L, L)]
        cnt, last = plsc.scan_count(v)
        b = plsc.load_gather(woff_v, [v])
        pos_v[k, pl.ds(i * L, L)] = b + cnt - 1
        plsc.store_scatter(woff_v, [v], b + cnt, mask=last)
        return 0

      lax.fori_loop(0, SCH // L, vec_body, 0)
      pltpu.async_copy(tgt_v.at[k], out_hbm.at[pos_v.at[k]], sems[k])
    return 0

  lax.fori_loop(0, NG, group_body, 0)
  for k in range(KR):
    pltpu.make_async_copy(
        tgt_v.at[k], out_hbm.at[pos_v.at[k]], sems[k]).wait()


@jax.jit
def _crs_neighbor(edge_index):
  src = edge_index[0].astype(i32)
  tgt = edge_index[1].astype(i32)
  hist, psum = _hist_kernel(src)
  splits_body, woff = _offsets_kernel(hist, psum)
  nbr = _scatter_kernel(src, tgt, woff)
  splits = jnp.concatenate(
      [jnp.zeros((1,), i32), splits_body[:N]]).astype(jnp.int64)
  return nbr.astype(jnp.int64), splits


def kernel(edge_index, length):
  del length
  return _crs_neighbor(edge_index)


# speedup vs baseline: 4.0400x; 3.2214x over previous
import functools

import jax
import jax.numpy as jnp
from jax import lax
from jax.experimental import pallas as pl
from jax.experimental.pallas import tpu as pltpu
from jax.experimental.pallas import tpu_sc as plsc

E = 6_400_000
N = 100_000
NC = 2
NS = 16
NW = NC * NS
EW = E // NW
NB = 3_136
NPAD = NB * NW
CH = 4_000
NCH = EW // CH
CB = 784
L = 16

W = 4_000
NWIN = EW // W
NRB = 98
NRBP = 112
BW = 65_536
LASTB = E - (NRB - 1) * BW
MAXU = 384
SPU = NWIN * (W // L + NRBP)
TOTU = NW * SPU
TOTUP = TOTU + 4 * MAXU
TRASH = TOTU
RCH = 128

_mesh = plsc.VectorSubcoreMesh(core_axis_name="c", subcore_axis_name="s")
_params = pltpu.CompilerParams(
    needs_layout_passes=False, use_tc_tiling_on_sc=False)

i32 = jnp.int32


def _wid():
  return lax.axis_index("s") * NC + lax.axis_index("c")


@functools.partial(
    pl.kernel,
    out_type=(
        jax.ShapeDtypeStruct((NW, NPAD), i32),
        jax.ShapeDtypeStruct((NW, NW), i32),
    ),
    mesh=_mesh,
    compiler_params=_params,
    scratch_types=[
        pltpu.VMEM((NPAD,), i32),
        pltpu.VMEM((CH,), i32),
        pltpu.VMEM((NW,), i32),
    ],
)
def _hist_kernel(src_hbm, hist_hbm, psum_hbm, hist_v, src_v, psum_v):
  wid = _wid()

  @plsc.parallel_loop(0, NPAD // L, unroll=8)
  def _(i):
    hist_v[pl.ds(i * L, L)] = jnp.zeros((L,), i32)

  def chunk_body(ci, _):
    base = wid * EW + ci * CH
    pltpu.sync_copy(src_hbm.at[pl.ds(base, CH)], src_v)

    def vec_body(i, _):
      v = src_v[pl.ds(i * L, L)]
      cnt, last = plsc.scan_count(v)
      plsc.addupdate_scatter(hist_v, [v], cnt, mask=last)
      return 0

    lax.fori_loop(0, CH // L, vec_body, 0)
    return 0

  lax.fori_loop(0, NCH, chunk_body, 0)

  lane0 = lax.iota(i32, L) == 0
  for r in range(NW):
    def sum_body(j, acc):
      return acc + hist_v[pl.ds(r * NB + j * L, L)]

    acc = lax.fori_loop(0, NB // L, sum_body, jnp.zeros((L,), i32))
    total = jnp.sum(acc)
    plsc.store_scatter(
        psum_v, [jnp.full((L,), r, i32)], jnp.full((L,), total, i32),
        mask=lane0)

  pltpu.sync_copy(hist_v, hist_hbm.at[wid])
  pltpu.sync_copy(psum_v, psum_hbm.at[wid])


@functools.partial(
    pl.kernel,
    out_type=(
        jax.ShapeDtypeStruct((NPAD,), i32),
        jax.ShapeDtypeStruct((NW, NPAD), i32),
    ),
    mesh=_mesh,
    compiler_params=_params,
    scratch_types=[
        pltpu.VMEM((NW, NW), i32),
        pltpu.VMEM((NW, CB), i32),
        pltpu.VMEM((NW, CB), i32),
        pltpu.VMEM((CB,), i32),
    ],
)
def _offsets_kernel(hist_hbm, psum_hbm, splits_hbm, woff_hbm, psum_v, hcol_v,
                    woff_v, spl_v):
  wid = _wid()
  pltpu.sync_copy(psum_hbm, psum_v)

  acc0 = jnp.zeros((L,), i32)
  acc1 = jnp.zeros((L,), i32)
  for w2 in range(NW):
    acc0 = acc0 + psum_v[w2, pl.ds(0, L)]
    acc1 = acc1 + psum_v[w2, pl.ds(L, L)]
  iota = lax.iota(i32, L)
  zero = jnp.zeros((L,), i32)
  base = jnp.sum(jnp.where(iota < wid, acc0, zero)) + jnp.sum(
      jnp.where(iota + L < wid, acc1, zero))

  def sub_chunk(k, carry):
    off = wid * NB + k * CB
    pltpu.sync_copy(hist_hbm.at[:, pl.ds(off, CB)], hcol_v)

    def vec_body(i, c):
      tot = jnp.zeros((L,), i32)
      for w2 in range(NW):
        tot = tot + hcol_v[w2, pl.ds(i * L, L)]
      incl = plsc.cumsum(tot) + jnp.full((L,), c, i32)
      spl_v[pl.ds(i * L, L)] = incl
      run = incl - tot
      for w2 in range(NW):
        woff_v[w2, pl.ds(i * L, L)] = run
        run = run + hcol_v[w2, pl.ds(i * L, L)]
      return c + jnp.sum(tot)

    carry = lax.fori_loop(0, CB // L, vec_body, carry)
    pltpu.sync_copy(woff_v, woff_hbm.at[:, pl.ds(off, CB)])
    pltpu.sync_copy(spl_v, splits_hbm.at[pl.ds(off, CB)])
    return carry

  lax.fori_loop(0, NB // CB, sub_chunk, base)


@functools.partial(
    pl.kernel,
    out_type=(
        jax.ShapeDtypeStruct((TOTUP, L), i32),
        jax.ShapeDtypeStruct((TOTUP, L), i32),
        jax.ShapeDtypeStruct((NW, NRBP + L), i32),
    ),
    mesh=_mesh,
    compiler_params=_params,
    scratch_types=[
        pltpu.VMEM((NPAD,), i32),
        pltpu.VMEM((W,), i32),
        pltpu.VMEM((W,), i32),
        pltpu.VMEM((W,), i32),
        pltpu.VMEM((MAXU, L), i32),
        pltpu.VMEM((MAXU, L), i32),
        pltpu.VMEM((NRBP,), i32),
        pltpu.VMEM((NRBP,), i32),
        pltpu.VMEM((NRBP,), i32),
        pltpu.VMEM((NRBP,), i32),
        pltpu.VMEM((NRBP,), i32),
        pltpu.VMEM((NRBP + L,), i32),
        pltpu.VMEM((MAXU,), i32),
        pltpu.VMEM((MAXU // RCH, RCH), i32),
    ],
)
def _bucketize_kernel(src_hbm, tgt_hbm, woff_hbm, bpos_hbm, btgt_hbm,
                      sstart_hbm, woff_v, src_v, tgt_v, posw_v, spos_v,
                      stgt_v, h_v, pbu_v, loffw_v, loffs_v, curg_v, sst_v,
                      rid_v, uidx_v):
  wid = _wid()
  zero16 = jnp.zeros((L,), i32)
  iota = lax.iota(i32, L)

  def zero_tab(tab):
    for q in range(NRBP // L):
      tab[pl.ds(q * L, L)] = zero16

  def window_positions(win, need_tgt):
    base = wid * EW + win * W
    pltpu.sync_copy(src_hbm.at[pl.ds(base, W)], src_v)
    if need_tgt:
      pltpu.sync_copy(tgt_hbm.at[pl.ds(base, W)], tgt_v)
    zero_tab(h_v)

    def vec_body(i, _):
      v = src_v[pl.ds(i * L, L)]
      cnt, last = plsc.scan_count(v)
      b = plsc.load_gather(woff_v, [v])
      pos = b + cnt - 1
      plsc.store_scatter(woff_v, [v], b + cnt, mask=last)
      posw_v[pl.ds(i * L, L)] = pos
      r = lax.shift_right_logical(pos, 16)
      cnt2, last2 = plsc.scan_count(r)
      plsc.addupdate_scatter(h_v, [r], cnt2, mask=last2)
      return 0

    lax.fori_loop(0, W // L, vec_body, 0)

  def piece_units(q):
    h = h_v[pl.ds(q * L, L)]
    return jnp.maximum(lax.shift_right_logical(h + 15, 4), 1)

  pltpu.sync_copy(woff_hbm.at[wid], woff_v)
  zero_tab(pbu_v)

  def sweep1_body(win, _):
    window_positions(win, need_tgt=False)
    for q in range(NRBP // L):
      pbu_v[pl.ds(q * L, L)] += piece_units(q)
    return 0

  lax.fori_loop(0, NWIN, sweep1_body, 0)

  carry = jnp.zeros((), i32)
  for q in range(NRBP // L):
    pu = pbu_v[pl.ds(q * L, L)]
    incl = plsc.cumsum(pu) + jnp.full((L,), carry, i32)
    curg_v[pl.ds(q * L, L)] = incl - pu + wid * SPU
    sst_v[pl.ds(q * L, L)] = incl - pu + wid * SPU
    carry = carry + jnp.sum(pu)
  sst_v[pl.ds(NRBP, L)] = jnp.full((L,), carry, i32) + wid * SPU
  pltpu.sync_copy(sst_v, sstart_hbm.at[wid])

  pltpu.sync_copy(woff_hbm.at[wid], woff_v)
  sent16 = jnp.full((L,), -1, i32)

  def sweep2_body(win, _):
    window_positions(win, need_tgt=True)

    c2 = jnp.zeros((), i32)
    for q in range(NRBP // L):
      pu = piece_units(q)
      incl = plsc.cumsum(pu) + jnp.full((L,), c2, i32)
      loffw_v[pl.ds(q * L, L)] = (incl - pu) * L
      loffs_v[pl.ds(q * L, L)] = (incl - pu) * L
      c2 = c2 + jnp.sum(pu)
    total_u = c2

    def pre_body(u, _):
      spos_v[u, :] = sent16
      return 0

    lax.fori_loop(0, MAXU, pre_body, 0)

    def sort_body(i, _):
      pos = posw_v[pl.ds(i * L, L)]
      tgt = tgt_v[pl.ds(i * L, L)]
      r = lax.shift_right_logical(pos, 16)
      cnt, last = plsc.scan_count(r)
      bw = plsc.load_gather(loffw_v, [r])
      idx = bw + cnt - 1
      plsc.store_scatter(loffw_v, [r], bw + cnt, mask=last)
      ir = lax.shift_right_logical(idx, 4)
      ic = jnp.bitwise_and(idx, 15)
      plsc.store_scatter(spos_v, [ir, ic], pos)
      plsc.store_scatter(stgt_v, [ir, ic], tgt)
      return 0

    lax.fori_loop(0, W // L, sort_body, 0)

    def rz_body(u, _):
      rid_v[pl.ds(u * L, L)] = zero16
      return 0

    lax.fori_loop(0, MAXU // L, rz_body, 0)
    for q in range(NRBP // L):
      rq = iota + q * L
      starts_u = lax.shift_right_logical(loffs_v[pl.ds(q * L, L)], 4)
      plsc.store_scatter(rid_v, [starts_u], rq,
                         mask=rq < jnp.full((L,), NRB, i32))
    mcarry = jnp.zeros((), i32)
    for u in range(MAXU // L):
      m = plsc.cummax(rid_v[pl.ds(u * L, L)])
      m = jnp.maximum(m, jnp.full((L,), mcarry, i32))
      rid_v[pl.ds(u * L, L)] = m
      mcarry = jnp.max(m)

    for u in range(MAXU // L):
      uu = iota + u * L
      r = rid_v[pl.ds(u * L, L)]
      start_u = lax.shift_right_logical(plsc.load_gather(loffs_v, [r]), 4)
      dst = plsc.load_gather(curg_v, [r]) + uu - start_u
      uidx_v[u // 8, pl.ds((u % 8) * L, L)] = jnp.where(
          uu < jnp.full((L,), total_u, i32), dst,
          jnp.full((L,), TRASH, i32) + uu)

    for q in range(NRBP // L):
      curg_v[pl.ds(q * L, L)] += piece_units(q)

    for j in range(MAXU // RCH):
      pltpu.sync_copy(spos_v.at[pl.ds(j * RCH, RCH)],
                      bpos_hbm.at[uidx_v.at[j]])
      pltpu.sync_copy(stgt_v.at[pl.ds(j * RCH, RCH)],
                      btgt_hbm.at[uidx_v.at[j]])
    return 0

  lax.fori_loop(0, NWIN, sweep2_body, 0)


@functools.partial(
    pl.kernel,
    out_type=jax.ShapeDtypeStruct((E,), i32),
    mesh=_mesh,
    compiler_params=_params,
    scratch_types=[
        pltpu.VMEM((BW,), i32),
        pltpu.VMEM((RCH, L), i32),
        pltpu.VMEM((RCH, L), i32),
        pltpu.VMEM((NW, NRBP + L), i32),
    ],
)
def _place_kernel(bpos_hbm, btgt_hbm, sstart_hbm, out_hbm, stage_v, pos_v,
                  tgt_v, sst_v):
  wid = _wid()
  pltpu.sync_copy(sstart_hbm, sst_v)

  def do_bucket(b, flush_words):
    bbase = b * BW

    def span_body(w2, _):
      row = sst_v[w2, pl.ds(b, L)]
      su = row[0]
      eu = row[1]

      def chunk_body(cu, _):
        au = su + cu * RCH
        pltpu.sync_copy(bpos_hbm.at[pl.ds(au, RCH)], pos_v)
        pltpu.sync_copy(btgt_hbm.at[pl.ds(au, RCH)], tgt_v)
        rem = jnp.minimum(eu - su - cu * RCH, RCH)

        def unit_body(u, _):
          pos = pos_v[u, :]
          tgt = tgt_v[u, :]
          ok = pos >= jnp.zeros((L,), i32)
          rel = pos - jnp.full((L,), bbase, i32)
          plsc.store_scatter(stage_v, [rel], tgt, mask=ok)
          return 0

        lax.fori_loop(0, rem, unit_body, 0)
        return 0

      nchunk = lax.shift_right_logical(eu - su + RCH - 1, 7)
      lax.fori_loop(0, nchunk, chunk_body, 0)
      return 0

    lax.fori_loop(0, NW, span_body, 0)
    pltpu.sync_copy(stage_v.at[pl.ds(0, flush_words)],
                    out_hbm.at[pl.ds(b * BW, flush_words)])

  for j in range(3):
    do_bucket(wid + 32 * j, BW)

  @pl.when(wid == 0)
  def _():
    do_bucket(jnp.full((), 96, i32), BW)

  @pl.when(wid == 1)
  def _():
    do_bucket(jnp.full((), 97, i32), LASTB)


@jax.jit
def _crs_neighbor(edge_index):
  src = edge_index[0].astype(i32)
  tgt = edge_index[1].astype(i32)
  hist, psum = _hist_kernel(src)
  splits_body, woff = _offsets_kernel(hist, psum)
  bpos, btgt, sstart = _bucketize_kernel(src, tgt, woff)
  nbr = _place_kernel(bpos, btgt, sstart)
  splits = jnp.concatenate(
      [jnp.zeros((1,), i32), splits_body[:N]]).astype(jnp.int64)
  return nbr.astype(jnp.int64), splits


def kernel(edge_index, length):
  del length
  return _crs_neighbor(edge_index)


# speedup vs baseline: 5.2621x; 1.3025x over previous
import functools

import jax
import jax.numpy as jnp
from jax import lax
from jax.experimental import pallas as pl
from jax.experimental.pallas import tpu as pltpu
from jax.experimental.pallas import tpu_sc as plsc

E = 6_400_000
N = 100_000
NC = 2
NS = 16
NW = NC * NS
EW = E // NW
NB = 3_136
NPAD = NB * NW
CH = 4_000
NCH = EW // CH
CB = 784
L = 16

W = 4_000
NWIN = EW // W
NRB = 98
NRBP = 112
BW = 65_536
LASTB = E - (NRB - 1) * BW
MAXU = 384
SPU = NWIN * (W // L + NRBP)
TOTU = NW * SPU
TOTUP = TOTU + 4 * MAXU
TRASH = TOTU
RCH = 128

_mesh = plsc.VectorSubcoreMesh(core_axis_name="c", subcore_axis_name="s")
_params = pltpu.CompilerParams(
    needs_layout_passes=False, use_tc_tiling_on_sc=False)

i32 = jnp.int32


def _wid():
  return lax.axis_index("s") * NC + lax.axis_index("c")


@functools.partial(
    pl.kernel,
    out_type=(
        jax.ShapeDtypeStruct((NW, NPAD), i32),
        jax.ShapeDtypeStruct((NW, NW), i32),
    ),
    mesh=_mesh,
    compiler_params=_params,
    scratch_types=[
        pltpu.VMEM((NPAD,), i32),
        pltpu.VMEM((CH,), i32),
        pltpu.VMEM((NW,), i32),
    ],
)
def _hist_kernel(src_hbm, hist_hbm, psum_hbm, hist_v, src_v, psum_v):
  wid = _wid()

  @plsc.parallel_loop(0, NPAD // L, unroll=8)
  def _(i):
    hist_v[pl.ds(i * L, L)] = jnp.zeros((L,), i32)

  def chunk_body(ci, _):
    base = wid * EW + ci * CH
    pltpu.sync_copy(src_hbm.at[pl.ds(base, CH)], src_v)

    @plsc.parallel_loop(0, CH // L, unroll=8)
    def _(i):
      v = src_v[pl.ds(i * L, L)]
      cnt, last = plsc.scan_count(v)
      plsc.addupdate_scatter(hist_v, [v], cnt, mask=last)

    return 0

  lax.fori_loop(0, NCH, chunk_body, 0)

  lane0 = lax.iota(i32, L) == 0
  for r in range(NW):
    def sum_body(j, acc):
      return acc + hist_v[pl.ds(r * NB + j * L, L)]

    acc = lax.fori_loop(0, NB // L, sum_body, jnp.zeros((L,), i32))
    total = jnp.sum(acc)
    plsc.store_scatter(
        psum_v, [jnp.full((L,), r, i32)], jnp.full((L,), total, i32),
        mask=lane0)

  pltpu.sync_copy(hist_v, hist_hbm.at[wid])
  pltpu.sync_copy(psum_v, psum_hbm.at[wid])


@functools.partial(
    pl.kernel,
    out_type=(
        jax.ShapeDtypeStruct((NPAD,), i32),
        jax.ShapeDtypeStruct((NW, NPAD), i32),
    ),
    mesh=_mesh,
    compiler_params=_params,
    scratch_types=[
        pltpu.VMEM((NW, NW), i32),
        pltpu.VMEM((NW, CB), i32),
        pltpu.VMEM((NW, CB), i32),
        pltpu.VMEM((CB,), i32),
    ],
)
def _offsets_kernel(hist_hbm, psum_hbm, splits_hbm, woff_hbm, psum_v, hcol_v,
                    woff_v, spl_v):
  wid = _wid()
  pltpu.sync_copy(psum_hbm, psum_v)

  acc0 = jnp.zeros((L,), i32)
  acc1 = jnp.zeros((L,), i32)
  for w2 in range(NW):
    acc0 = acc0 + psum_v[w2, pl.ds(0, L)]
    acc1 = acc1 + psum_v[w2, pl.ds(L, L)]
  iota = lax.iota(i32, L)
  zero = jnp.zeros((L,), i32)
  base = jnp.sum(jnp.where(iota < wid, acc0, zero)) + jnp.sum(
      jnp.where(iota + L < wid, acc1, zero))

  def sub_chunk(k, carry):
    off = wid * NB + k * CB
    pltpu.sync_copy(hist_hbm.at[:, pl.ds(off, CB)], hcol_v)

    def vec_body(i, c):
      tot = jnp.zeros((L,), i32)
      for w2 in range(NW):
        tot = tot + hcol_v[w2, pl.ds(i * L, L)]
      incl = plsc.cumsum(tot) + jnp.full((L,), c, i32)
      spl_v[pl.ds(i * L, L)] = incl
      run = incl - tot
      for w2 in range(NW):
        woff_v[w2, pl.ds(i * L, L)] = run
        run = run + hcol_v[w2, pl.ds(i * L, L)]
      return c + jnp.sum(tot)

    carry = lax.fori_loop(0, CB // L, vec_body, carry)
    pltpu.sync_copy(woff_v, woff_hbm.at[:, pl.ds(off, CB)])
    pltpu.sync_copy(spl_v, splits_hbm.at[pl.ds(off, CB)])
    return carry

  lax.fori_loop(0, NB // CB, sub_chunk, base)


@functools.partial(
    pl.kernel,
    out_type=(
        jax.ShapeDtypeStruct((TOTUP, L), i32),
        jax.ShapeDtypeStruct((TOTUP, L), i32),
        jax.ShapeDtypeStruct((NW, NRBP + L), i32),
    ),
    mesh=_mesh,
    compiler_params=_params,
    scratch_types=[
        pltpu.VMEM((NPAD,), i32),
        pltpu.VMEM((W,), i32),
        pltpu.VMEM((W,), i32),
        pltpu.VMEM((W,), i32),
        pltpu.VMEM((W,), i32),
        pltpu.VMEM((MAXU, L), i32),
        pltpu.VMEM((MAXU, L), i32),
        pltpu.VMEM((NRBP,), i32),
        pltpu.VMEM((NRBP,), i32),
        pltpu.VMEM((NRBP,), i32),
        pltpu.VMEM((NRBP,), i32),
        pltpu.VMEM((NRBP,), i32),
        pltpu.VMEM((NRBP + L,), i32),
        pltpu.VMEM((MAXU,), i32),
        pltpu.VMEM((MAXU // RCH, RCH), i32),
        pltpu.SemaphoreType.DMA,
    ],
)
def _bucketize_kernel(src_hbm, tgt_hbm, woff_hbm, bpos_hbm, btgt_hbm,
                      sstart_hbm, woff_v, src_v, tgt_v, posw_v, cntw_v,
                      spos_v, stgt_v, h_v, pbu_v, loffw_v, loffs_v, curg_v,
                      sst_v, rid_v, uidx_v, fsem):
  wid = _wid()
  zero16 = jnp.zeros((L,), i32)
  iota = lax.iota(i32, L)

  def zero_tab(tab):
    for q in range(NRBP // L):
      tab[pl.ds(q * L, L)] = zero16

  def window_positions(win, need_tgt):
    base = wid * EW + win * W
    pltpu.sync_copy(src_hbm.at[pl.ds(base, W)], src_v)
    if need_tgt:
      pltpu.sync_copy(tgt_hbm.at[pl.ds(base, W)], tgt_v)
    zero_tab(h_v)

    def vec_body(i, _):
      v = src_v[pl.ds(i * L, L)]
      cnt, last = plsc.scan_count(v)
      b = plsc.load_gather(woff_v, [v])
      pos = b + cnt - 1
      plsc.store_scatter(woff_v, [v], b + cnt, mask=last)
      posw_v[pl.ds(i * L, L)] = pos
      r = lax.shift_right_logical(pos, 16)
      cnt2, last2 = plsc.scan_count(r)
      plsc.addupdate_scatter(h_v, [r], cnt2, mask=last2)
      cntw_v[pl.ds(i * L, L)] = cnt2 + last2.astype(i32) * 256
      return 0

    lax.fori_loop(0, W // L, vec_body, 0)

  def piece_units(q):
    h = h_v[pl.ds(q * L, L)]
    return jnp.maximum(lax.shift_right_logical(h + 15, 4), 1)

  pltpu.sync_copy(woff_hbm.at[wid], woff_v)
  zero_tab(pbu_v)

  def sweep1_body(win, _):
    window_positions(win, need_tgt=False)
    for q in range(NRBP // L):
      pbu_v[pl.ds(q * L, L)] += piece_units(q)
    return 0

  lax.fori_loop(0, NWIN, sweep1_body, 0)

  carry = jnp.zeros((), i32)
  for q in range(NRBP // L):
    pu = pbu_v[pl.ds(q * L, L)]
    incl = plsc.cumsum(pu) + jnp.full((L,), carry, i32)
    curg_v[pl.ds(q * L, L)] = incl - pu + wid * SPU
    sst_v[pl.ds(q * L, L)] = incl - pu + wid * SPU
    carry = carry + jnp.sum(pu)
  sst_v[pl.ds(NRBP, L)] = jnp.full((L,), carry, i32) + wid * SPU
  pltpu.sync_copy(sst_v, sstart_hbm.at[wid])

  pltpu.sync_copy(woff_hbm.at[wid], woff_v)
  sent16 = jnp.full((L,), -1, i32)

  def sweep2_body(win, _):
    window_positions(win, need_tgt=True)

    c2 = jnp.zeros((), i32)
    for q in range(NRBP // L):
      pu = piece_units(q)
      incl = plsc.cumsum(pu) + jnp.full((L,), c2, i32)
      loffw_v[pl.ds(q * L, L)] = (incl - pu) * L
      loffs_v[pl.ds(q * L, L)] = (incl - pu) * L
      c2 = c2 + jnp.sum(pu)
    total_u = c2

    @pl.when(win > 0)
    def _():
      for j in range(MAXU // RCH):
        pltpu.make_async_copy(spos_v.at[pl.ds(j * RCH, RCH)],
                              bpos_hbm.at[uidx_v.at[j]], fsem).wait()
        pltpu.make_async_copy(stgt_v.at[pl.ds(j * RCH, RCH)],
                              btgt_hbm.at[uidx_v.at[j]], fsem).wait()

    @plsc.parallel_loop(0, MAXU, unroll=8)
    def _(u):
      spos_v[u, :] = sent16

    def sort_body(i, _):
      pos = posw_v[pl.ds(i * L, L)]
      tgt = tgt_v[pl.ds(i * L, L)]
      cl = cntw_v[pl.ds(i * L, L)]
      cnt = jnp.bitwise_and(cl, 255)
      last = lax.shift_right_logical(cl, 8) > jnp.zeros((L,), i32)
      r = lax.shift_right_logical(pos, 16)
      bw = plsc.load_gather(loffw_v, [r])
      idx = bw + cnt - 1
      plsc.store_scatter(loffw_v, [r], bw + cnt, mask=last)
      ir = lax.shift_right_logical(idx, 4)
      ic = jnp.bitwise_and(idx, 15)
      plsc.store_scatter(spos_v, [ir, ic], pos)
      plsc.store_scatter(stgt_v, [ir, ic], tgt)
      return 0

    lax.fori_loop(0, W // L, sort_body, 0)

    def rz_body(u, _):
      rid_v[pl.ds(u * L, L)] = zero16
      return 0

    lax.fori_loop(0, MAXU // L, rz_body, 0)
    for q in range(NRBP // L):
      rq = iota + q * L
      starts_u = lax.shift_right_logical(loffs_v[pl.ds(q * L, L)], 4)
      plsc.store_scatter(rid_v, [starts_u], rq,
                         mask=rq < jnp.full((L,), NRB, i32))
    mcarry = jnp.zeros((), i32)
    for u in range(MAXU // L):
      m = plsc.cummax(rid_v[pl.ds(u * L, L)])
      m = jnp.maximum(m, jnp.full((L,), mcarry, i32))
      rid_v[pl.ds(u * L, L)] = m
      mcarry = jnp.max(m)

    for u in range(MAXU // L):
      uu = iota + u * L
      r = rid_v[pl.ds(u * L, L)]
      start_u = lax.shift_right_logical(plsc.load_gather(loffs_v, [r]), 4)
      dst = plsc.load_gather(curg_v, [r]) + uu - start_u
      uidx_v[u // 8, pl.ds((u % 8) * L, L)] = jnp.where(
          uu < jnp.full((L,), total_u, i32), dst,
          jnp.full((L,), TRASH, i32) + uu)

    for q in range(NRBP // L):
      curg_v[pl.ds(q * L, L)] += piece_units(q)

    for j in range(MAXU // RCH):
      pltpu.async_copy(spos_v.at[pl.ds(j * RCH, RCH)],
                       bpos_hbm.at[uidx_v.at[j]], fsem)
      pltpu.async_copy(stgt_v.at[pl.ds(j * RCH, RCH)],
                       btgt_hbm.at[uidx_v.at[j]], fsem)
    return 0

  lax.fori_loop(0, NWIN, sweep2_body, 0)
  for j in range(MAXU // RCH):
    pltpu.make_async_copy(spos_v.at[pl.ds(j * RCH, RCH)],
                          bpos_hbm.at[uidx_v.at[j]], fsem).wait()
    pltpu.make_async_copy(stgt_v.at[pl.ds(j * RCH, RCH)],
                          btgt_hbm.at[uidx_v.at[j]], fsem).wait()


@functools.partial(
    pl.kernel,
    out_type=jax.ShapeDtypeStruct((E,), i32),
    mesh=_mesh,
    compiler_params=_params,
    scratch_types=[
        pltpu.VMEM((BW,), i32),
        pltpu.VMEM((RCH, L), i32),
        pltpu.VMEM((RCH, L), i32),
        pltpu.VMEM((NW, NRBP + L), i32),
    ],
)
def _place_kernel(bpos_hbm, btgt_hbm, sstart_hbm, out_hbm, stage_v, pos_v,
                  tgt_v, sst_v):
  wid = _wid()
  pltpu.sync_copy(sstart_hbm, sst_v)

  def do_bucket(b, flush_words):
    bbase = b * BW

    def span_body(w2, _):
      row = sst_v[w2, pl.ds(b, L)]
      su = row[0]
      eu = row[1]

      def chunk_body(cu, _):
        au = su + cu * RCH
        pltpu.sync_copy(bpos_hbm.at[pl.ds(au, RCH)], pos_v)
        pltpu.sync_copy(btgt_hbm.at[pl.ds(au, RCH)], tgt_v)
        rem = jnp.minimum(eu - su - cu * RCH, RCH)

        @plsc.parallel_loop(0, rem, unroll=4)
        def _(u):
          pos = pos_v[u, :]
          tgt = tgt_v[u, :]
          ok = pos >= jnp.zeros((L,), i32)
          rel = pos - jnp.full((L,), bbase, i32)
          plsc.store_scatter(stage_v, [rel], tgt, mask=ok)

        return 0

      nchunk = lax.shift_right_logical(eu - su + RCH - 1, 7)
      lax.fori_loop(0, nchunk, chunk_body, 0)
      return 0

    lax.fori_loop(0, NW, span_body, 0)
    pltpu.sync_copy(stage_v.at[pl.ds(0, flush_words)],
                    out_hbm.at[pl.ds(b * BW, flush_words)])

  for j in range(3):
    do_bucket(wid + 32 * j, BW)

  @pl.when(wid == 0)
  def _():
    do_bucket(jnp.full((), 96, i32), BW)

  @pl.when(wid == 1)
  def _():
    do_bucket(jnp.full((), 97, i32), LASTB)


@jax.jit
def _crs_neighbor(edge_index):
  src = edge_index[0].astype(i32)
  tgt = edge_index[1].astype(i32)
  hist, psum = _hist_kernel(src)
  splits_body, woff = _offsets_kernel(hist, psum)
  bpos, btgt, sstart = _bucketize_kernel(src, tgt, woff)
  nbr = _place_kernel(bpos, btgt, sstart)
  splits = jnp.concatenate(
      [jnp.zeros((1,), i32), splits_body[:N]]).astype(jnp.int64)
  return nbr.astype(jnp.int64), splits


def kernel(edge_index, length):
  del length
  return _crs_neighbor(edge_index)


# speedup vs baseline: 5.5522x; 1.0551x over previous
import functools

import jax
import jax.numpy as jnp
from jax import lax
from jax.experimental import pallas as pl
from jax.experimental.pallas import tpu as pltpu
from jax.experimental.pallas import tpu_sc as plsc

E = 6_400_000
N = 100_000
NC = 2
NS = 16
NW = NC * NS
EW = E // NW
NB = 3_136
NPAD = NB * NW
CH = 4_000
NCH = EW // CH
CB = 784
L = 16

W = 4_000
NWIN = EW // W
NRB = 98
NRBP = 112
BW = 65_536
LASTB = E - (NRB - 1) * BW
MAXU = 384
SPU = NWIN * (W // L + NRBP)
TOTU = NW * SPU
TOTUP = TOTU + 4 * MAXU
TRASH = TOTU
RCH = 128

_mesh = plsc.VectorSubcoreMesh(core_axis_name="c", subcore_axis_name="s")
_params = pltpu.CompilerParams(
    needs_layout_passes=False, use_tc_tiling_on_sc=False)

i32 = jnp.int32


def _wid():
  return lax.axis_index("s") * NC + lax.axis_index("c")


@functools.partial(
    pl.kernel,
    out_type=(
        jax.ShapeDtypeStruct((NW, NPAD), i32),
        jax.ShapeDtypeStruct((NW, NW), i32),
    ),
    mesh=_mesh,
    compiler_params=_params,
    scratch_types=[
        pltpu.VMEM((NPAD,), i32),
        pltpu.VMEM((CH,), i32),
        pltpu.VMEM((NW,), i32),
    ],
)
def _hist_kernel(src_hbm, hist_hbm, psum_hbm, hist_v, src_v, psum_v):
  wid = _wid()

  @plsc.parallel_loop(0, NPAD // L, unroll=8)
  def _(i):
    hist_v[pl.ds(i * L, L)] = jnp.zeros((L,), i32)

  def chunk_body(ci, _):
    base = wid * EW + ci * CH
    pltpu.sync_copy(src_hbm.at[pl.ds(base, CH)], src_v)

    @plsc.parallel_loop(0, CH // L, unroll=8)
    def _(i):
      v = src_v[pl.ds(i * L, L)]
      cnt, last = plsc.scan_count(v)
      plsc.addupdate_scatter(hist_v, [v], cnt, mask=last)

    return 0

  lax.fori_loop(0, NCH, chunk_body, 0)

  lane0 = lax.iota(i32, L) == 0
  for r in range(NW):
    def sum_body(j, acc):
      return acc + hist_v[pl.ds(r * NB + j * L, L)]

    acc = lax.fori_loop(0, NB // L, sum_body, jnp.zeros((L,), i32))
    total = jnp.sum(acc)
    plsc.store_scatter(
        psum_v, [jnp.full((L,), r, i32)], jnp.full((L,), total, i32),
        mask=lane0)

  pltpu.sync_copy(hist_v, hist_hbm.at[wid])
  pltpu.sync_copy(psum_v, psum_hbm.at[wid])


@functools.partial(
    pl.kernel,
    out_type=(
        jax.ShapeDtypeStruct((NPAD,), i32),
        jax.ShapeDtypeStruct((NW, NPAD), i32),
    ),
    mesh=_mesh,
    compiler_params=_params,
    scratch_types=[
        pltpu.VMEM((NW, NW), i32),
        pltpu.VMEM((NW, CB), i32),
        pltpu.VMEM((NW, CB), i32),
        pltpu.VMEM((CB,), i32),
    ],
)
def _offsets_kernel(hist_hbm, psum_hbm, splits_hbm, woff_hbm, psum_v, hcol_v,
                    woff_v, spl_v):
  wid = _wid()
  pltpu.sync_copy(psum_hbm, psum_v)

  acc0 = jnp.zeros((L,), i32)
  acc1 = jnp.zeros((L,), i32)
  for w2 in range(NW):
    acc0 = acc0 + psum_v[w2, pl.ds(0, L)]
    acc1 = acc1 + psum_v[w2, pl.ds(L, L)]
  iota = lax.iota(i32, L)
  zero = jnp.zeros((L,), i32)
  base = jnp.sum(jnp.where(iota < wid, acc0, zero)) + jnp.sum(
      jnp.where(iota + L < wid, acc1, zero))

  def sub_chunk(k, carry):
    off = wid * NB + k * CB
    pltpu.sync_copy(hist_hbm.at[:, pl.ds(off, CB)], hcol_v)

    def vec_body(i, c):
      tot = jnp.zeros((L,), i32)
      for w2 in range(NW):
        tot = tot + hcol_v[w2, pl.ds(i * L, L)]
      incl = plsc.cumsum(tot) + jnp.full((L,), c, i32)
      spl_v[pl.ds(i * L, L)] = incl
      run = incl - tot
      for w2 in range(NW):
        woff_v[w2, pl.ds(i * L, L)] = run
        run = run + hcol_v[w2, pl.ds(i * L, L)]
      return c + jnp.sum(tot)

    carry = lax.fori_loop(0, CB // L, vec_body, carry)
    pltpu.sync_copy(woff_v, woff_hbm.at[:, pl.ds(off, CB)])
    pltpu.sync_copy(spl_v, splits_hbm.at[pl.ds(off, CB)])
    return carry

  lax.fori_loop(0, NB // CB, sub_chunk, base)


@functools.partial(
    pl.kernel,
    out_type=(
        jax.ShapeDtypeStruct((TOTUP, L), i32),
        jax.ShapeDtypeStruct((TOTUP, L), i32),
        jax.ShapeDtypeStruct((NW, NRBP + L), i32),
    ),
    mesh=_mesh,
    compiler_params=_params,
    scratch_types=[
        pltpu.VMEM((NPAD,), i32),
        pltpu.VMEM((W,), i32),
        pltpu.VMEM((W,), i32),
        pltpu.VMEM((W,), i32),
        pltpu.VMEM((W,), i32),
        pltpu.VMEM((MAXU, L), i32),
        pltpu.VMEM((MAXU, L), i32),
        pltpu.VMEM((NRBP,), i32),
        pltpu.VMEM((NRBP,), i32),
        pltpu.VMEM((NRBP,), i32),
        pltpu.VMEM((NRBP,), i32),
        pltpu.VMEM((NRBP,), i32),
        pltpu.VMEM((NRBP + L,), i32),
        pltpu.VMEM((MAXU,), i32),
        pltpu.VMEM((MAXU // RCH, RCH), i32),
        pltpu.SemaphoreType.DMA,
        pltpu.SemaphoreType.DMA,
        pltpu.SemaphoreType.DMA,
    ],
)
def _bucketize_kernel(src_hbm, tgt_hbm, woff_hbm, bpos_hbm, btgt_hbm,
                      sstart_hbm, woff_v, src_v, tgt_v, posw_v, cntw_v,
                      spos_v, stgt_v, h_v, pbu_v, loffw_v, loffs_v, curg_v,
                      sst_v, rid_v, uidx_v, fsem, ssem, tsem):
  wid = _wid()
  zero16 = jnp.zeros((L,), i32)
  iota = lax.iota(i32, L)

  def zero_tab(tab):
    for q in range(NRBP // L):
      tab[pl.ds(q * L, L)] = zero16

  def src_copy(win):
    base = wid * EW + win * W
    return pltpu.make_async_copy(src_hbm.at[pl.ds(base, W)], src_v, ssem)

  def tgt_copy(win):
    base = wid * EW + win * W
    return pltpu.make_async_copy(tgt_hbm.at[pl.ds(base, W)], tgt_v, tsem)

  def window_positions(win):
    zero_tab(h_v)

    def vec_body(i, _):
      v = src_v[pl.ds(i * L, L)]
      cnt, last = plsc.scan_count(v)
      b = plsc.load_gather(woff_v, [v])
      pos = b + cnt - 1
      plsc.store_scatter(woff_v, [v], b + cnt, mask=last)
      posw_v[pl.ds(i * L, L)] = pos
      r = lax.shift_right_logical(pos, 16)
      cnt2, last2 = plsc.scan_count(r)
      plsc.addupdate_scatter(h_v, [r], cnt2, mask=last2)
      cntw_v[pl.ds(i * L, L)] = cnt2 + last2.astype(i32) * 256
      return 0

    lax.fori_loop(0, W // L, vec_body, 0)

  def piece_units(q):
    h = h_v[pl.ds(q * L, L)]
    return jnp.maximum(lax.shift_right_logical(h + 15, 4), 1)

  pltpu.sync_copy(woff_hbm.at[wid], woff_v)
  zero_tab(pbu_v)

  src_copy(0).start()

  def sweep1_body(win, _):
    src_copy(win).wait()
    window_positions(win)

    @pl.when(win + 1 < NWIN)
    def _():
      src_copy(win + 1).start()

    for q in range(NRBP // L):
      pbu_v[pl.ds(q * L, L)] += piece_units(q)
    return 0

  lax.fori_loop(0, NWIN, sweep1_body, 0)

  carry = jnp.zeros((), i32)
  for q in range(NRBP // L):
    pu = pbu_v[pl.ds(q * L, L)]
    incl = plsc.cumsum(pu) + jnp.full((L,), carry, i32)
    curg_v[pl.ds(q * L, L)] = incl - pu + wid * SPU
    sst_v[pl.ds(q * L, L)] = incl - pu + wid * SPU
    carry = carry + jnp.sum(pu)
  sst_v[pl.ds(NRBP, L)] = jnp.full((L,), carry, i32) + wid * SPU
  pltpu.sync_copy(sst_v, sstart_hbm.at[wid])

  pltpu.sync_copy(woff_hbm.at[wid], woff_v)
  sent16 = jnp.full((L,), -1, i32)
  src_copy(0).start()

  def sweep2_body(win, _):
    tgt_copy(win).start()
    src_copy(win).wait()
    window_positions(win)

    @pl.when(win + 1 < NWIN)
    def _():
      src_copy(win + 1).start()

    c2 = jnp.zeros((), i32)
    for q in range(NRBP // L):
      pu = piece_units(q)
      incl = plsc.cumsum(pu) + jnp.full((L,), c2, i32)
      loffw_v[pl.ds(q * L, L)] = (incl - pu) * L
      loffs_v[pl.ds(q * L, L)] = (incl - pu) * L
      c2 = c2 + jnp.sum(pu)
    total_u = c2

    @pl.when(win > 0)
    def _():
      for j in range(MAXU // RCH):
        pltpu.make_async_copy(spos_v.at[pl.ds(j * RCH, RCH)],
                              bpos_hbm.at[uidx_v.at[j]], fsem).wait()
        pltpu.make_async_copy(stgt_v.at[pl.ds(j * RCH, RCH)],
                              btgt_hbm.at[uidx_v.at[j]], fsem).wait()

    @plsc.parallel_loop(0, MAXU, unroll=8)
    def _(u):
      spos_v[u, :] = sent16

    tgt_copy(win).wait()

    def sort_body(i, _):
      pos = posw_v[pl.ds(i * L, L)]
      tgt = tgt_v[pl.ds(i * L, L)]
      cl = cntw_v[pl.ds(i * L, L)]
      cnt = jnp.bitwise_and(cl, 255)
      last = lax.shift_right_logical(cl, 8) > jnp.zeros((L,), i32)
      r = lax.shift_right_logical(pos, 16)
      bw = plsc.load_gather(loffw_v, [r])
      idx = bw + cnt - 1
      plsc.store_scatter(loffw_v, [r], bw + cnt, mask=last)
      ir = lax.shift_right_logical(idx, 4)
      ic = jnp.bitwise_and(idx, 15)
      plsc.store_scatter(spos_v, [ir, ic], pos)
      plsc.store_scatter(stgt_v, [ir, ic], tgt)
      return 0

    lax.fori_loop(0, W // L, sort_body, 0)

    def rz_body(u, _):
      rid_v[pl.ds(u * L, L)] = zero16
      return 0

    lax.fori_loop(0, MAXU // L, rz_body, 0)
    for q in range(NRBP // L):
      rq = iota + q * L
      starts_u = lax.shift_right_logical(loffs_v[pl.ds(q * L, L)], 4)
      plsc.store_scatter(rid_v, [starts_u], rq,
                         mask=rq < jnp.full((L,), NRB, i32))
    mcarry = jnp.zeros((), i32)
    for u in range(MAXU // L):
      m = plsc.cummax(rid_v[pl.ds(u * L, L)])
      m = jnp.maximum(m, jnp.full((L,), mcarry, i32))
      rid_v[pl.ds(u * L, L)] = m
      mcarry = jnp.max(m)

    for u in range(MAXU // L):
      uu = iota + u * L
      r = rid_v[pl.ds(u * L, L)]
      start_u = lax.shift_right_logical(plsc.load_gather(loffs_v, [r]), 4)
      dst = plsc.load_gather(curg_v, [r]) + uu - start_u
      uidx_v[u // 8, pl.ds((u % 8) * L, L)] = jnp.where(
          uu < jnp.full((L,), total_u, i32), dst,
          jnp.full((L,), TRASH, i32) + uu)

    for q in range(NRBP // L):
      curg_v[pl.ds(q * L, L)] += piece_units(q)

    for j in range(MAXU // RCH):
      pltpu.async_copy(spos_v.at[pl.ds(j * RCH, RCH)],
                       bpos_hbm.at[uidx_v.at[j]], fsem)
      pltpu.async_copy(stgt_v.at[pl.ds(j * RCH, RCH)],
                       btgt_hbm.at[uidx_v.at[j]], fsem)
    return 0

  lax.fori_loop(0, NWIN, sweep2_body, 0)
  for j in range(MAXU // RCH):
    pltpu.make_async_copy(spos_v.at[pl.ds(j * RCH, RCH)],
                          bpos_hbm.at[uidx_v.at[j]], fsem).wait()
    pltpu.make_async_copy(stgt_v.at[pl.ds(j * RCH, RCH)],
                          btgt_hbm.at[uidx_v.at[j]], fsem).wait()


@functools.partial(
    pl.kernel,
    out_type=jax.ShapeDtypeStruct((E,), i32),
    mesh=_mesh,
    compiler_params=_params,
    scratch_types=[
        pltpu.VMEM((BW,), i32),
        pltpu.VMEM((RCH, L), i32),
        pltpu.VMEM((RCH, L), i32),
        pltpu.VMEM((NW, NRBP + L), i32),
    ],
)
def _place_kernel(bpos_hbm, btgt_hbm, sstart_hbm, out_hbm, stage_v, pos_v,
                  tgt_v, sst_v):
  wid = _wid()
  pltpu.sync_copy(sstart_hbm, sst_v)

  def do_bucket(b, flush_words):
    bbase = b * BW

    def span_body(w2, _):
      row = sst_v[w2, pl.ds(b, L)]
      su = row[0]
      eu = row[1]

      def chunk_body(cu, _):
        au = su + cu * RCH
        pltpu.sync_copy(bpos_hbm.at[pl.ds(au, RCH)], pos_v)
        pltpu.sync_copy(btgt_hbm.at[pl.ds(au, RCH)], tgt_v)
        rem = jnp.minimum(eu - su - cu * RCH, RCH)

        @plsc.parallel_loop(0, rem, unroll=4)
        def _(u):
          pos = pos_v[u, :]
          tgt = tgt_v[u, :]
          ok = pos >= jnp.zeros((L,), i32)
          rel = pos - jnp.full((L,), bbase, i32)
          plsc.store_scatter(stage_v, [rel], tgt, mask=ok)

        return 0

      nchunk = lax.shift_right_logical(eu - su + RCH - 1, 7)
      lax.fori_loop(0, nchunk, chunk_body, 0)
      return 0

    lax.fori_loop(0, NW, span_body, 0)
    pltpu.sync_copy(stage_v.at[pl.ds(0, flush_words)],
                    out_hbm.at[pl.ds(b * BW, flush_words)])

  for j in range(3):
    do_bucket(wid + 32 * j, BW)

  @pl.when(wid == 0)
  def _():
    do_bucket(jnp.full((), 96, i32), BW)

  @pl.when(wid == 1)
  def _():
    do_bucket(jnp.full((), 97, i32), LASTB)


@jax.jit
def _crs_neighbor(edge_index):
  src = edge_index[0].astype(i32)
  tgt = edge_index[1].astype(i32)
  hist, psum = _hist_kernel(src)
  splits_body, woff = _offsets_kernel(hist, psum)
  bpos, btgt, sstart = _bucketize_kernel(src, tgt, woff)
  nbr = _place_kernel(bpos, btgt, sstart)
  splits = jnp.concatenate(
      [jnp.zeros((1,), i32), splits_body[:N]]).astype(jnp.int64)
  return nbr.astype(jnp.int64), splits


def kernel(edge_index, length):
  del length
  return _crs_neighbor(edge_index)


# speedup vs baseline: 6.2042x; 1.1174x over previous
import functools

import jax
import jax.numpy as jnp
from jax import lax
from jax.experimental import pallas as pl
from jax.experimental.pallas import tpu as pltpu
from jax.experimental.pallas import tpu_sc as plsc

E = 6_400_000
N = 100_000
NC = 2
NS = 16
NW = NC * NS
EW = E // NW
NB = 3_136
NPAD = NB * NW
CH = 4_000
NCH = EW // CH
CB = 784
L = 16

W = 4_000
NWIN = EW // W
NRB = 98
NRBP = 112
BW = 65_536
LASTB = E - (NRB - 1) * BW
MAXU = 384
SPU = NWIN * (W // L + NRBP)
TOTU = NW * SPU
TOTUP = TOTU + 4 * MAXU
TRASH = TOTU
RCH = 128

_mesh = plsc.VectorSubcoreMesh(core_axis_name="c", subcore_axis_name="s")
_params = pltpu.CompilerParams(
    needs_layout_passes=False, use_tc_tiling_on_sc=False)

i32 = jnp.int32


def _wid():
  return lax.axis_index("s") * NC + lax.axis_index("c")


@functools.partial(
    pl.kernel,
    out_type=(
        jax.ShapeDtypeStruct((NW, NPAD), i32),
        jax.ShapeDtypeStruct((NW, NW), i32),
    ),
    mesh=_mesh,
    compiler_params=_params,
    scratch_types=[
        pltpu.VMEM((NPAD,), i32),
        pltpu.VMEM((CH,), i32),
        pltpu.VMEM((NW,), i32),
    ],
)
def _hist_kernel(src_hbm, hist_hbm, psum_hbm, hist_v, src_v, psum_v):
  wid = _wid()

  @plsc.parallel_loop(0, NPAD // L, unroll=8)
  def _(i):
    hist_v[pl.ds(i * L, L)] = jnp.zeros((L,), i32)

  def chunk_body(ci, _):
    base = wid * EW + ci * CH
    pltpu.sync_copy(src_hbm.at[pl.ds(base, CH)], src_v)

    @plsc.parallel_loop(0, CH // L, unroll=8)
    def _(i):
      v = src_v[pl.ds(i * L, L)]
      cnt, last = plsc.scan_count(v)
      plsc.addupdate_scatter(hist_v, [v], cnt, mask=last)

    return 0

  lax.fori_loop(0, NCH, chunk_body, 0)

  lane0 = lax.iota(i32, L) == 0
  for r in range(NW):
    def sum_body(j, acc):
      return acc + hist_v[pl.ds(r * NB + j * L, L)]

    acc = lax.fori_loop(0, NB // L, sum_body, jnp.zeros((L,), i32))
    total = jnp.sum(acc)
    plsc.store_scatter(
        psum_v, [jnp.full((L,), r, i32)], jnp.full((L,), total, i32),
        mask=lane0)

  pltpu.sync_copy(hist_v, hist_hbm.at[wid])
  pltpu.sync_copy(psum_v, psum_hbm.at[wid])


@functools.partial(
    pl.kernel,
    out_type=(
        jax.ShapeDtypeStruct((NPAD,), i32),
        jax.ShapeDtypeStruct((NW, NPAD), i32),
    ),
    mesh=_mesh,
    compiler_params=_params,
    scratch_types=[
        pltpu.VMEM((NW, NW), i32),
        pltpu.VMEM((NW, CB), i32),
        pltpu.VMEM((NW, CB), i32),
        pltpu.VMEM((CB,), i32),
    ],
)
def _offsets_kernel(hist_hbm, psum_hbm, splits_hbm, woff_hbm, psum_v, hcol_v,
                    woff_v, spl_v):
  wid = _wid()
  pltpu.sync_copy(psum_hbm, psum_v)

  acc0 = jnp.zeros((L,), i32)
  acc1 = jnp.zeros((L,), i32)
  for w2 in range(NW):
    acc0 = acc0 + psum_v[w2, pl.ds(0, L)]
    acc1 = acc1 + psum_v[w2, pl.ds(L, L)]
  iota = lax.iota(i32, L)
  zero = jnp.zeros((L,), i32)
  base = jnp.sum(jnp.where(iota < wid, acc0, zero)) + jnp.sum(
      jnp.where(iota + L < wid, acc1, zero))

  def sub_chunk(k, carry):
    off = wid * NB + k * CB
    pltpu.sync_copy(hist_hbm.at[:, pl.ds(off, CB)], hcol_v)

    def vec_body(i, c):
      tot = jnp.zeros((L,), i32)
      for w2 in range(NW):
        tot = tot + hcol_v[w2, pl.ds(i * L, L)]
      incl = plsc.cumsum(tot) + jnp.full((L,), c, i32)
      spl_v[pl.ds(i * L, L)] = incl
      run = incl - tot
      for w2 in range(NW):
        woff_v[w2, pl.ds(i * L, L)] = run
        run = run + hcol_v[w2, pl.ds(i * L, L)]
      return c + jnp.sum(tot)

    carry = lax.fori_loop(0, CB // L, vec_body, carry)
    pltpu.sync_copy(woff_v, woff_hbm.at[:, pl.ds(off, CB)])
    pltpu.sync_copy(spl_v, splits_hbm.at[pl.ds(off, CB)])
    return carry

  lax.fori_loop(0, NB // CB, sub_chunk, base)


@functools.partial(
    pl.kernel,
    out_type=(
        jax.ShapeDtypeStruct((TOTUP, L), i32),
        jax.ShapeDtypeStruct((TOTUP, L), i32),
        jax.ShapeDtypeStruct((NW, NRBP + L), i32),
    ),
    mesh=_mesh,
    compiler_params=_params,
    scratch_types=[
        pltpu.VMEM((NPAD,), i32),
        pltpu.VMEM((W,), i32),
        pltpu.VMEM((W,), i32),
        pltpu.VMEM((W,), i32),
        pltpu.VMEM((W,), i32),
        pltpu.VMEM((MAXU, L), i32),
        pltpu.VMEM((MAXU, L), i32),
        pltpu.VMEM((NRBP,), i32),
        pltpu.VMEM((NRBP,), i32),
        pltpu.VMEM((NRBP,), i32),
        pltpu.VMEM((NRBP,), i32),
        pltpu.VMEM((NRBP,), i32),
        pltpu.VMEM((NRBP + L,), i32),
        pltpu.VMEM((MAXU,), i32),
        pltpu.VMEM((MAXU // RCH, RCH), i32),
        pltpu.SemaphoreType.DMA,
        pltpu.SemaphoreType.DMA,
        pltpu.SemaphoreType.DMA,
    ],
)
def _bucketize_kernel(src_hbm, tgt_hbm, woff_hbm, bpos_hbm, btgt_hbm,
                      sstart_hbm, woff_v, src_v, tgt_v, posw_v, cntw_v,
                      spos_v, stgt_v, h_v, pbu_v, loffw_v, loffs_v, curg_v,
                      sst_v, rid_v, uidx_v, fsem, ssem, tsem):
  wid = _wid()
  zero16 = jnp.zeros((L,), i32)
  iota = lax.iota(i32, L)

  def zero_tab(tab):
    for q in range(NRBP // L):
      tab[pl.ds(q * L, L)] = zero16

  def src_copy(win):
    base = wid * EW + win * W
    return pltpu.make_async_copy(src_hbm.at[pl.ds(base, W)], src_v, ssem)

  def tgt_copy(win):
    base = wid * EW + win * W
    return pltpu.make_async_copy(tgt_hbm.at[pl.ds(base, W)], tgt_v, tsem)

  def window_positions(win):
    zero_tab(h_v)

    def vec_body(i, _):
      v = src_v[pl.ds(i * L, L)]
      cnt, last = plsc.scan_count(v)
      b = plsc.load_gather(woff_v, [v])
      pos = b + cnt - 1
      plsc.store_scatter(woff_v, [v], b + cnt, mask=last)
      posw_v[pl.ds(i * L, L)] = pos
      r = lax.shift_right_logical(pos, 16)
      cnt2, last2 = plsc.scan_count(r)
      plsc.addupdate_scatter(h_v, [r], cnt2, mask=last2)
      cntw_v[pl.ds(i * L, L)] = cnt2 + last2.astype(i32) * 256
      return 0

    lax.fori_loop(0, W // L, vec_body, 0)

  def piece_units(q):
    h = h_v[pl.ds(q * L, L)]
    return jnp.maximum(lax.shift_right_logical(h + 15, 4), 1)

  pltpu.sync_copy(woff_hbm.at[wid], woff_v)
  zero_tab(pbu_v)

  src_copy(0).start()

  def sweep1_body(win, _):
    src_copy(win).wait()
    window_positions(win)

    @pl.when(win + 1 < NWIN)
    def _():
      src_copy(win + 1).start()

    for q in range(NRBP // L):
      pbu_v[pl.ds(q * L, L)] += piece_units(q)
    return 0

  lax.fori_loop(0, NWIN, sweep1_body, 0)

  carry = jnp.zeros((), i32)
  for q in range(NRBP // L):
    pu = pbu_v[pl.ds(q * L, L)]
    incl = plsc.cumsum(pu) + jnp.full((L,), carry, i32)
    curg_v[pl.ds(q * L, L)] = incl - pu + wid * SPU
    sst_v[pl.ds(q * L, L)] = incl - pu + wid * SPU
    carry = carry + jnp.sum(pu)
  sst_v[pl.ds(NRBP, L)] = jnp.full((L,), carry, i32) + wid * SPU
  pltpu.sync_copy(sst_v, sstart_hbm.at[wid])

  pltpu.sync_copy(woff_hbm.at[wid], woff_v)
  sent16 = jnp.full((L,), -1, i32)
  src_copy(0).start()

  def sweep2_body(win, _):
    tgt_copy(win).start()
    src_copy(win).wait()
    window_positions(win)

    @pl.when(win + 1 < NWIN)
    def _():
      src_copy(win + 1).start()

    c2 = jnp.zeros((), i32)
    for q in range(NRBP // L):
      pu = piece_units(q)
      incl = plsc.cumsum(pu) + jnp.full((L,), c2, i32)
      loffw_v[pl.ds(q * L, L)] = (incl - pu) * L
      loffs_v[pl.ds(q * L, L)] = (incl - pu) * L
      c2 = c2 + jnp.sum(pu)
    total_u = c2

    @pl.when(win > 0)
    def _():
      for j in range(MAXU // RCH):
        pltpu.make_async_copy(spos_v.at[pl.ds(j * RCH, RCH)],
                              bpos_hbm.at[uidx_v.at[j]], fsem).wait()
        pltpu.make_async_copy(stgt_v.at[pl.ds(j * RCH, RCH)],
                              btgt_hbm.at[uidx_v.at[j]], fsem).wait()

    @plsc.parallel_loop(0, MAXU, unroll=8)
    def _(u):
      spos_v[u, :] = sent16

    tgt_copy(win).wait()

    def sort_body(i, _):
      pos = posw_v[pl.ds(i * L, L)]
      tgt = tgt_v[pl.ds(i * L, L)]
      cl = cntw_v[pl.ds(i * L, L)]
      cnt = jnp.bitwise_and(cl, 255)
      last = lax.shift_right_logical(cl, 8) > jnp.zeros((L,), i32)
      r = lax.shift_right_logical(pos, 16)
      bw = plsc.load_gather(loffw_v, [r])
      idx = bw + cnt - 1
      plsc.store_scatter(loffw_v, [r], bw + cnt, mask=last)
      ir = lax.shift_right_logical(idx, 4)
      ic = jnp.bitwise_and(idx, 15)
      plsc.store_scatter(spos_v, [ir, ic], pos)
      plsc.store_scatter(stgt_v, [ir, ic], tgt)
      return 0

    lax.fori_loop(0, W // L, sort_body, 0)

    def rz_body(u, _):
      rid_v[pl.ds(u * L, L)] = zero16
      return 0

    lax.fori_loop(0, MAXU // L, rz_body, 0)
    for q in range(NRBP // L):
      rq = iota + q * L
      starts_u = lax.shift_right_logical(loffs_v[pl.ds(q * L, L)], 4)
      plsc.store_scatter(rid_v, [starts_u], rq,
                         mask=rq < jnp.full((L,), NRB, i32))
    mcarry = jnp.zeros((), i32)
    for u in range(MAXU // L):
      m = plsc.cummax(rid_v[pl.ds(u * L, L)])
      m = jnp.maximum(m, jnp.full((L,), mcarry, i32))
      rid_v[pl.ds(u * L, L)] = m
      mcarry = jnp.max(m)

    for u in range(MAXU // L):
      uu = iota + u * L
      r = rid_v[pl.ds(u * L, L)]
      start_u = lax.shift_right_logical(plsc.load_gather(loffs_v, [r]), 4)
      dst = plsc.load_gather(curg_v, [r]) + uu - start_u
      uidx_v[u // 8, pl.ds((u % 8) * L, L)] = jnp.where(
          uu < jnp.full((L,), total_u, i32), dst,
          jnp.full((L,), TRASH, i32) + uu)

    for q in range(NRBP // L):
      curg_v[pl.ds(q * L, L)] += piece_units(q)

    for j in range(MAXU // RCH):
      pltpu.async_copy(spos_v.at[pl.ds(j * RCH, RCH)],
                       bpos_hbm.at[uidx_v.at[j]], fsem)
      pltpu.async_copy(stgt_v.at[pl.ds(j * RCH, RCH)],
                       btgt_hbm.at[uidx_v.at[j]], fsem)
    return 0

  lax.fori_loop(0, NWIN, sweep2_body, 0)
  for j in range(MAXU // RCH):
    pltpu.make_async_copy(spos_v.at[pl.ds(j * RCH, RCH)],
                          bpos_hbm.at[uidx_v.at[j]], fsem).wait()
    pltpu.make_async_copy(stgt_v.at[pl.ds(j * RCH, RCH)],
                          btgt_hbm.at[uidx_v.at[j]], fsem).wait()


@functools.partial(
    pl.kernel,
    out_type=jax.ShapeDtypeStruct((E,), i32),
    mesh=_mesh,
    compiler_params=_params,
    scratch_types=[
        pltpu.VMEM((BW,), i32),
        pltpu.VMEM((RCH, L), i32),
        pltpu.VMEM((RCH, L), i32),
        pltpu.VMEM((RCH, L), i32),
        pltpu.VMEM((RCH, L), i32),
        pltpu.VMEM((NW, NRBP + L), i32),
        pltpu.SemaphoreType.DMA,
        pltpu.SemaphoreType.DMA,
    ],
)
def _place_kernel(bpos_hbm, btgt_hbm, sstart_hbm, out_hbm, stage_v, pos_a,
                  tgt_a, pos_b, tgt_b, sst_v, sem_a, sem_b):
  wid = _wid()
  pltpu.sync_copy(sstart_hbm, sst_v)

  def span_of(w2, b):
    row = sst_v[w2, pl.ds(b, L)]
    return row[0], row[1]

  def fire(pos_v, tgt_v, au, sem):
    pltpu.make_async_copy(bpos_hbm.at[pl.ds(au, RCH)], pos_v, sem).start()
    pltpu.make_async_copy(btgt_hbm.at[pl.ds(au, RCH)], tgt_v, sem).start()

  def drain(pos_v, tgt_v, au, sem):
    pltpu.make_async_copy(bpos_hbm.at[pl.ds(au, RCH)], pos_v, sem).wait()
    pltpu.make_async_copy(btgt_hbm.at[pl.ds(au, RCH)], tgt_v, sem).wait()

  def do_bucket(b, flush_words):
    bbase = b * BW

    def process_chunk(pos_v, tgt_v, rem):
      @plsc.parallel_loop(0, rem, unroll=4)
      def _(u):
        pos = pos_v[u, :]
        tgt = tgt_v[u, :]
        ok = pos >= jnp.zeros((L,), i32)
        rel = pos - jnp.full((L,), bbase, i32)
        plsc.store_scatter(stage_v, [rel], tgt, mask=ok)

    def process_span(pos_v, tgt_v, sem, su, eu):
      drain(pos_v, tgt_v, su, sem)
      process_chunk(pos_v, tgt_v, jnp.minimum(eu - su, RCH))

      def chunk_body(cu, _):
        au = su + cu * RCH
        pltpu.sync_copy(bpos_hbm.at[pl.ds(au, RCH)], pos_v)
        pltpu.sync_copy(btgt_hbm.at[pl.ds(au, RCH)], tgt_v)
        process_chunk(pos_v, tgt_v, jnp.minimum(eu - au, RCH))
        return 0

      nchunk = lax.shift_right_logical(eu - su + RCH - 1, 7)
      lax.fori_loop(1, nchunk, chunk_body, 0)

    sua0, _ = span_of(0, b)
    fire(pos_a, tgt_a, sua0, sem_a)

    def pair_body(p, _):
      w2a = 2 * p
      sua, eua = span_of(w2a, b)
      sub, eub = span_of(w2a + 1, b)
      fire(pos_b, tgt_b, sub, sem_b)
      process_span(pos_a, tgt_a, sem_a, sua, eua)

      @pl.when(p + 1 < NW // 2)
      def _():
        sun, _ = span_of(w2a + 2, b)
        fire(pos_a, tgt_a, sun, sem_a)

      process_span(pos_b, tgt_b, sem_b, sub, eub)
      return 0

    lax.fori_loop(0, NW // 2, pair_body, 0)
    pltpu.sync_copy(stage_v.at[pl.ds(0, flush_words)],
                    out_hbm.at[pl.ds(b * BW, flush_words)])

  for j in range(3):
    do_bucket(wid + 32 * j, BW)

  @pl.when(wid == 0)
  def _():
    do_bucket(jnp.full((), 96, i32), BW)

  @pl.when(wid == 1)
  def _():
    do_bucket(jnp.full((), 97, i32), LASTB)


@jax.jit
def _crs_neighbor(edge_index):
  src = edge_index[0].astype(i32)
  tgt = edge_index[1].astype(i32)
  hist, psum = _hist_kernel(src)
  splits_body, woff = _offsets_kernel(hist, psum)
  bpos, btgt, sstart = _bucketize_kernel(src, tgt, woff)
  nbr = _place_kernel(bpos, btgt, sstart)
  splits = jnp.concatenate(
      [jnp.zeros((1,), i32), splits_body[:N]]).astype(jnp.int64)
  return nbr.astype(jnp.int64), splits


def kernel(edge_index, length):
  del length
  return _crs_neighbor(edge_index)


# speedup vs baseline: 6.9554x; 1.1211x over previous
import functools

import jax
import jax.numpy as jnp
from jax import lax
from jax.experimental import pallas as pl
from jax.experimental.pallas import tpu as pltpu
from jax.experimental.pallas import tpu_sc as plsc

E = 6_400_000
N = 100_000
NC = 2
NS = 16
NW = NC * NS
EW = E // NW
NB = 3_136
NPAD = NB * NW
CH = 4_000
NCH = EW // CH
CB = 784
L = 16

W = 4_000
NWIN = EW // W
NRB = 98
NRBP = 112
BW = 65_536
LASTB = E - (NRB - 1) * BW
MAXU = 384
ALLOC_SLACK = 2 * NWIN - 2
SPU = EW // L + NRBP * ALLOC_SLACK
TOTU = NW * SPU
TOTUP = TOTU + 4 * MAXU
TRASH = TOTU
RCH = 128

_mesh = plsc.VectorSubcoreMesh(core_axis_name="c", subcore_axis_name="s")
_params = pltpu.CompilerParams(
    needs_layout_passes=False, use_tc_tiling_on_sc=False)

i32 = jnp.int32


def _wid():
  return lax.axis_index("s") * NC + lax.axis_index("c")


def _take16(vec, idx):
  return lax.gather(
      vec, idx[:, None],
      lax.GatherDimensionNumbers(offset_dims=(), collapsed_slice_dims=(0,),
                                 start_index_map=(0,)),
      slice_sizes=(1,), mode=lax.GatherScatterMode.PROMISE_IN_BOUNDS)


@functools.partial(
    pl.kernel,
    out_type=(
        jax.ShapeDtypeStruct((NW, NPAD), i32),
        jax.ShapeDtypeStruct((NW, NW), i32),
    ),
    mesh=_mesh,
    compiler_params=_params,
    scratch_types=[
        pltpu.VMEM((NPAD,), i32),
        pltpu.VMEM((CH,), i32),
        pltpu.VMEM((NW,), i32),
    ],
)
def _hist_kernel(src_hbm, hist_hbm, psum_hbm, hist_v, src_v, psum_v):
  wid = _wid()

  @plsc.parallel_loop(0, NPAD // L, unroll=8)
  def _(i):
    hist_v[pl.ds(i * L, L)] = jnp.zeros((L,), i32)

  def chunk_body(ci, _):
    base = wid * EW + ci * CH
    pltpu.sync_copy(src_hbm.at[pl.ds(base, CH)], src_v)

    @plsc.parallel_loop(0, CH // L, unroll=8)
    def _(i):
      v = src_v[pl.ds(i * L, L)]
      cnt, last = plsc.scan_count(v)
      plsc.addupdate_scatter(hist_v, [v], cnt, mask=last)

    return 0

  lax.fori_loop(0, NCH, chunk_body, 0)

  lane0 = lax.iota(i32, L) == 0
  for r in range(NW):
    def sum_body(j, acc):
      return acc + hist_v[pl.ds(r * NB + j * L, L)]

    acc = lax.fori_loop(0, NB // L, sum_body, jnp.zeros((L,), i32))
    total = jnp.sum(acc)
    plsc.store_scatter(
        psum_v, [jnp.full((L,), r, i32)], jnp.full((L,), total, i32),
        mask=lane0)

  pltpu.sync_copy(hist_v, hist_hbm.at[wid])
  pltpu.sync_copy(psum_v, psum_hbm.at[wid])


@functools.partial(
    pl.kernel,
    out_type=(
        jax.ShapeDtypeStruct((NPAD,), i32),
        jax.ShapeDtypeStruct((NW, NPAD), i32),
    ),
    mesh=_mesh,
    compiler_params=_params,
    scratch_types=[
        pltpu.VMEM((NW, NW), i32),
        pltpu.VMEM((NW, CB), i32),
        pltpu.VMEM((NW, CB), i32),
        pltpu.VMEM((CB,), i32),
    ],
)
def _offsets_kernel(hist_hbm, psum_hbm, splits_hbm, woff_hbm, psum_v, hcol_v,
                    woff_v, spl_v):
  wid = _wid()
  pltpu.sync_copy(psum_hbm, psum_v)

  acc0 = jnp.zeros((L,), i32)
  acc1 = jnp.zeros((L,), i32)
  for w2 in range(NW):
    acc0 = acc0 + psum_v[w2, pl.ds(0, L)]
    acc1 = acc1 + psum_v[w2, pl.ds(L, L)]
  iota = lax.iota(i32, L)
  zero = jnp.zeros((L,), i32)
  base = jnp.sum(jnp.where(iota < wid, acc0, zero)) + jnp.sum(
      jnp.where(iota + L < wid, acc1, zero))

  def sub_chunk(k, carry):
    off = wid * NB + k * CB
    pltpu.sync_copy(hist_hbm.at[:, pl.ds(off, CB)], hcol_v)

    def vec_body(i, c):
      tot = jnp.zeros((L,), i32)
      for w2 in range(NW):
        tot = tot + hcol_v[w2, pl.ds(i * L, L)]
      incl = plsc.cumsum(tot) + jnp.full((L,), c, i32)
      spl_v[pl.ds(i * L, L)] = incl
      run = incl - tot
      for w2 in range(NW):
        woff_v[w2, pl.ds(i * L, L)] = run
        run = run + hcol_v[w2, pl.ds(i * L, L)]
      return c + jnp.sum(tot)

    carry = lax.fori_loop(0, CB // L, vec_body, carry)
    pltpu.sync_copy(woff_v, woff_hbm.at[:, pl.ds(off, CB)])
    pltpu.sync_copy(spl_v, splits_hbm.at[pl.ds(off, CB)])
    return carry

  lax.fori_loop(0, NB // CB, sub_chunk, base)


@functools.partial(
    pl.kernel,
    out_type=(
        jax.ShapeDtypeStruct((TOTUP, L), i32),
        jax.ShapeDtypeStruct((TOTUP, L), i32),
        jax.ShapeDtypeStruct((NW, NRBP + L), i32),
        jax.ShapeDtypeStruct((NW, NRBP + L), i32),
    ),
    mesh=_mesh,
    compiler_params=_params,
    scratch_types=[
        pltpu.VMEM((NPAD,), i32),
        pltpu.VMEM((W,), i32),
        pltpu.VMEM((W,), i32),
        pltpu.VMEM((W,), i32),
        pltpu.VMEM((W,), i32),
        pltpu.VMEM((MAXU, L), i32),
        pltpu.VMEM((MAXU, L), i32),
        pltpu.VMEM((NRBP,), i32),
        pltpu.VMEM((NRBP,), i32),
        pltpu.VMEM((NRBP,), i32),
        pltpu.VMEM((NRBP,), i32),
        pltpu.VMEM((NRBP,), i32),
        pltpu.VMEM((NRBP + L,), i32),
        pltpu.VMEM((MAXU,), i32),
        pltpu.VMEM((MAXU // RCH, RCH), i32),
        pltpu.SemaphoreType.DMA,
        pltpu.SemaphoreType.DMA,
        pltpu.SemaphoreType.DMA,
    ],
)
def _bucketize_kernel(src_hbm, tgt_hbm, woff_hbm, hist_hbm, bpos_hbm,
                      btgt_hbm, sstart_hbm, send_hbm, woff_v, src_v, tgt_v,
                      posw_v, cntw_v, spos_v, stgt_v, h_v, pbu_v, loffw_v,
                      loffs_v, curg_v, sst_v, rid_v, uidx_v, fsem, ssem,
                      tsem):
  wid = _wid()
  zero16 = jnp.zeros((L,), i32)
  iota = lax.iota(i32, L)

  def zero_tab(tab):
    for q in range(NRBP // L):
      tab[pl.ds(q * L, L)] = zero16

  def src_copy(win):
    base = wid * EW + win * W
    return pltpu.make_async_copy(src_hbm.at[pl.ds(base, W)], src_v, ssem)

  def tgt_copy(win):
    base = wid * EW + win * W
    return pltpu.make_async_copy(tgt_hbm.at[pl.ds(base, W)], tgt_v, tsem)

  def window_positions(win):
    zero_tab(h_v)

    def vec_body(i, _):
      v = src_v[pl.ds(i * L, L)]
      cnt, last = plsc.scan_count(v)
      b = plsc.load_gather(woff_v, [v])
      pos = b + cnt - 1
      plsc.store_scatter(woff_v, [v], b + cnt, mask=last)
      posw_v[pl.ds(i * L, L)] = pos
      r = lax.shift_right_logical(pos, 16)
      cnt2, last2 = plsc.scan_count(r)
      plsc.addupdate_scatter(h_v, [r], cnt2, mask=last2)
      cntw_v[pl.ds(i * L, L)] = cnt2 + last2.astype(i32) * 256
      return 0

    lax.fori_loop(0, W // L, vec_body, 0)

  def piece_units(q):
    h = h_v[pl.ds(q * L, L)]
    return jnp.maximum(lax.shift_right_logical(h + 15, 4), 1)

  pltpu.sync_copy(woff_hbm.at[wid], woff_v)
  zero_tab(pbu_v)

  def seg_add(keys, vals, valid):
    vz = jnp.where(valid, vals, jnp.zeros((L,), i32))
    s = plsc.cumsum(vz)
    cnt, last_m = plsc.scan_count(keys, mask=valid)
    prev_i = iota - cnt
    prev_s = _take16(s, jnp.maximum(prev_i, jnp.zeros((L,), i32)))
    prev_s = jnp.where(prev_i < jnp.zeros((L,), i32),
                       jnp.zeros((L,), i32), prev_s)
    plsc.addupdate_scatter(pbu_v, [keys], s - prev_s, mask=last_m)

  HCB = 3_136
  for hc in range(NPAD // HCB):
    pltpu.sync_copy(hist_hbm.at[wid, pl.ds(hc * HCB, HCB)],
                    posw_v.at[pl.ds(0, HCB)])

    def pb_body(i, _):
      wo = woff_v[pl.ds(hc * HCB + i * L, L)]
      c = posw_v[pl.ds(i * L, L)]
      r0 = lax.shift_right_logical(wo, 16)
      cap = jnp.full((L,), BW, i32) - jnp.bitwise_and(
          wo, jnp.full((L,), BW - 1, i32))
      a0 = jnp.minimum(c, cap)
      seg_add(r0, a0, jnp.full((L,), True))

      def spill_cond(st):
        rem, _ = st
        return jnp.any(rem > jnp.zeros((L,), i32))

      def spill_body(st):
        rem, rr = st
        a = jnp.minimum(rem, jnp.full((L,), BW, i32))
        seg_add(rr, a, rem > jnp.zeros((L,), i32))
        return rem - a, rr + jnp.ones((L,), i32)

      lax.while_loop(spill_cond, spill_body,
                     (c - a0, r0 + jnp.ones((L,), i32)))
      return 0

    lax.fori_loop(0, HCB // L, pb_body, 0)

  carry = jnp.zeros((), i32)
  for q in range(NRBP // L):
    pbw = pbu_v[pl.ds(q * L, L)]
    au = lax.shift_right_logical(pbw, 4) + jnp.full((L,), ALLOC_SLACK, i32)
    incl = plsc.cumsum(au) + jnp.full((L,), carry, i32)
    curg_v[pl.ds(q * L, L)] = incl - au + wid * SPU
    sst_v[pl.ds(q * L, L)] = incl - au + wid * SPU
    carry = carry + jnp.sum(au)
  sst_v[pl.ds(NRBP, L)] = jnp.full((L,), carry, i32) + wid * SPU
  pltpu.sync_copy(sst_v, sstart_hbm.at[wid])

  pltpu.sync_copy(woff_hbm.at[wid], woff_v)
  sent16 = jnp.full((L,), -1, i32)
  src_copy(0).start()

  def sweep2_body(win, _):
    tgt_copy(win).start()
    src_copy(win).wait()
    window_positions(win)

    @pl.when(win + 1 < NWIN)
    def _():
      src_copy(win + 1).start()

    c2 = jnp.zeros((), i32)
    for q in range(NRBP // L):
      pu = piece_units(q)
      incl = plsc.cumsum(pu) + jnp.full((L,), c2, i32)
      loffw_v[pl.ds(q * L, L)] = (incl - pu) * L
      loffs_v[pl.ds(q * L, L)] = (incl - pu) * L
      c2 = c2 + jnp.sum(pu)
    total_u = c2

    @pl.when(win > 0)
    def _():
      for j in range(MAXU // RCH):
        pltpu.make_async_copy(spos_v.at[pl.ds(j * RCH, RCH)],
                              bpos_hbm.at[uidx_v.at[j]], fsem).wait()
        pltpu.make_async_copy(stgt_v.at[pl.ds(j * RCH, RCH)],
                              btgt_hbm.at[uidx_v.at[j]], fsem).wait()

    @plsc.parallel_loop(0, MAXU, unroll=8)
    def _(u):
      spos_v[u, :] = sent16

    tgt_copy(win).wait()

    def sort_body(i, _):
      pos = posw_v[pl.ds(i * L, L)]
      tgt = tgt_v[pl.ds(i * L, L)]
      cl = cntw_v[pl.ds(i * L, L)]
      cnt = jnp.bitwise_and(cl, 255)
      last = lax.shift_right_logical(cl, 8) > jnp.zeros((L,), i32)
      r = lax.shift_right_logical(pos, 16)
      bw = plsc.load_gather(loffw_v, [r])
      idx = bw + cnt - 1
      plsc.store_scatter(loffw_v, [r], bw + cnt, mask=last)
      ir = lax.shift_right_logical(idx, 4)
      ic = jnp.bitwise_and(idx, 15)
      plsc.store_scatter(spos_v, [ir, ic], pos)
      plsc.store_scatter(stgt_v, [ir, ic], tgt)
      return 0

    lax.fori_loop(0, W // L, sort_body, 0)

    def rz_body(u, _):
      rid_v[pl.ds(u * L, L)] = zero16
      return 0

    lax.fori_loop(0, MAXU // L, rz_body, 0)
    for q in range(NRBP // L):
      rq = iota + q * L
      starts_u = lax.shift_right_logical(loffs_v[pl.ds(q * L, L)], 4)
      plsc.store_scatter(rid_v, [starts_u], rq,
                         mask=rq < jnp.full((L,), NRB, i32))
    mcarry = jnp.zeros((), i32)
    for u in range(MAXU // L):
      m = plsc.cummax(rid_v[pl.ds(u * L, L)])
      m = jnp.maximum(m, jnp.full((L,), mcarry, i32))
      rid_v[pl.ds(u * L, L)] = m
      mcarry = jnp.max(m)

    for u in range(MAXU // L):
      uu = iota + u * L
      r = rid_v[pl.ds(u * L, L)]
      start_u = lax.shift_right_logical(plsc.load_gather(loffs_v, [r]), 4)
      dst = plsc.load_gather(curg_v, [r]) + uu - start_u
      uidx_v[u // 8, pl.ds((u % 8) * L, L)] = jnp.where(
          uu < jnp.full((L,), total_u, i32), dst,
          jnp.full((L,), TRASH, i32) + uu)

    for q in range(NRBP // L):
      curg_v[pl.ds(q * L, L)] += piece_units(q)

    for j in range(MAXU // RCH):
      pltpu.async_copy(spos_v.at[pl.ds(j * RCH, RCH)],
                       bpos_hbm.at[uidx_v.at[j]], fsem)
      pltpu.async_copy(stgt_v.at[pl.ds(j * RCH, RCH)],
                       btgt_hbm.at[uidx_v.at[j]], fsem)
    return 0

  lax.fori_loop(0, NWIN, sweep2_body, 0)
  for j in range(MAXU // RCH):
    pltpu.make_async_copy(spos_v.at[pl.ds(j * RCH, RCH)],
                          bpos_hbm.at[uidx_v.at[j]], fsem).wait()
    pltpu.make_async_copy(stgt_v.at[pl.ds(j * RCH, RCH)],
                          btgt_hbm.at[uidx_v.at[j]], fsem).wait()
  for q in range(NRBP // L):
    sst_v[pl.ds(q * L, L)] = curg_v[pl.ds(q * L, L)]
  pltpu.sync_copy(sst_v, send_hbm.at[wid])


@functools.partial(
    pl.kernel,
    out_type=jax.ShapeDtypeStruct((E,), i32),
    mesh=_mesh,
    compiler_params=_params,
    scratch_types=[
        pltpu.VMEM((BW,), i32),
        pltpu.VMEM((RCH, L), i32),
        pltpu.VMEM((RCH, L), i32),
        pltpu.VMEM((RCH, L), i32),
        pltpu.VMEM((RCH, L), i32),
        pltpu.VMEM((NW, NRBP + L), i32),
        pltpu.VMEM((NW, NRBP + L), i32),
        pltpu.SemaphoreType.DMA,
        pltpu.SemaphoreType.DMA,
    ],
)
def _place_kernel(bpos_hbm, btgt_hbm, sstart_hbm, send_hbm, out_hbm, stage_v,
                  pos_a, tgt_a, pos_b, tgt_b, sst_v, sse_v, sem_a, sem_b):
  wid = _wid()
  pltpu.sync_copy(sstart_hbm, sst_v)
  pltpu.sync_copy(send_hbm, sse_v)

  def span_of(w2, b):
    su = sst_v[w2, pl.ds(b, L)][0]
    eu = sse_v[w2, pl.ds(b, L)][0]
    return su, eu

  def fire(pos_v, tgt_v, au, sem):
    pltpu.make_async_copy(bpos_hbm.at[pl.ds(au, RCH)], pos_v, sem).start()
    pltpu.make_async_copy(btgt_hbm.at[pl.ds(au, RCH)], tgt_v, sem).start()

  def drain(pos_v, tgt_v, au, sem):
    pltpu.make_async_copy(bpos_hbm.at[pl.ds(au, RCH)], pos_v, sem).wait()
    pltpu.make_async_copy(btgt_hbm.at[pl.ds(au, RCH)], tgt_v, sem).wait()

  def do_bucket(b, flush_words):
    bbase = b * BW

    def process_chunk(pos_v, tgt_v, rem):
      @plsc.parallel_loop(0, rem, unroll=4)
      def _(u):
        pos = pos_v[u, :]
        tgt = tgt_v[u, :]
        ok = pos >= jnp.zeros((L,), i32)
        rel = pos - jnp.full((L,), bbase, i32)
        plsc.store_scatter(stage_v, [rel], tgt, mask=ok)

    def process_span(pos_v, tgt_v, sem, su, eu):
      drain(pos_v, tgt_v, su, sem)
      process_chunk(pos_v, tgt_v, jnp.minimum(eu - su, RCH))

      def chunk_body(cu, _):
        au = su + cu * RCH
        pltpu.sync_copy(bpos_hbm.at[pl.ds(au, RCH)], pos_v)
        pltpu.sync_copy(btgt_hbm.at[pl.ds(au, RCH)], tgt_v)
        process_chunk(pos_v, tgt_v, jnp.minimum(eu - au, RCH))
        return 0

      nchunk = lax.shift_right_logical(eu - su + RCH - 1, 7)
      lax.fori_loop(1, nchunk, chunk_body, 0)

    sua0, _ = span_of(0, b)
    fire(pos_a, tgt_a, sua0, sem_a)

    def pair_body(p, _):
      w2a = 2 * p
      sua, eua = span_of(w2a, b)
      sub, eub = span_of(w2a + 1, b)
      fire(pos_b, tgt_b, sub, sem_b)
      process_span(pos_a, tgt_a, sem_a, sua, eua)

      @pl.when(p + 1 < NW // 2)
      def _():
        sun, _ = span_of(w2a + 2, b)
        fire(pos_a, tgt_a, sun, sem_a)

      process_span(pos_b, tgt_b, sem_b, sub, eub)
      return 0

    lax.fori_loop(0, NW // 2, pair_body, 0)
    pltpu.sync_copy(stage_v.at[pl.ds(0, flush_words)],
                    out_hbm.at[pl.ds(b * BW, flush_words)])

  for j in range(3):
    do_bucket(wid + 32 * j, BW)

  @pl.when(wid == 0)
  def _():
    do_bucket(jnp.full((), 96, i32), BW)

  @pl.when(wid == 1)
  def _():
    do_bucket(jnp.full((), 97, i32), LASTB)


@jax.jit
def _crs_neighbor(edge_index):
  src = edge_index[0].astype(i32)
  tgt = edge_index[1].astype(i32)
  hist, psum = _hist_kernel(src)
  splits_body, woff = _offsets_kernel(hist, psum)
  bpos, btgt, sstart, send = _bucketize_kernel(src, tgt, woff, hist)
  nbr = _place_kernel(bpos, btgt, sstart, send)
  splits = jnp.concatenate(
      [jnp.zeros((1,), i32), splits_body[:N]]).astype(jnp.int64)
  return nbr.astype(jnp.int64), splits


def kernel(edge_index, length):
  del length
  return _crs_neighbor(edge_index)


# speedup vs baseline: 7.1376x; 1.0262x over previous
import functools

import jax
import jax.numpy as jnp
from jax import lax
from jax.experimental import pallas as pl
from jax.experimental.pallas import tpu as pltpu
from jax.experimental.pallas import tpu_sc as plsc

E = 6_400_000
N = 100_000
NC = 2
NS = 16
NW = NC * NS
EW = E // NW
NB = 3_136
NPAD = NB * NW
CH = 4_000
NCH = EW // CH
CB = 784
L = 16

W = 4_000
NWIN = EW // W
NRB = 98
NRBP = 112
BW = 65_536
LASTB = E - (NRB - 1) * BW
MAXU = 384
ALLOC_SLACK = 2 * NWIN - 2
SPU = EW // L + NRBP * ALLOC_SLACK
TOTU = NW * SPU
TOTUP = TOTU + 4 * MAXU
TRASH = TOTU
RCH = 128

_mesh = plsc.VectorSubcoreMesh(core_axis_name="c", subcore_axis_name="s")
_params = pltpu.CompilerParams(
    needs_layout_passes=False, use_tc_tiling_on_sc=False)

i32 = jnp.int32


def _wid():
  return lax.axis_index("s") * NC + lax.axis_index("c")


def _take16(vec, idx):
  return lax.gather(
      vec, idx[:, None],
      lax.GatherDimensionNumbers(offset_dims=(), collapsed_slice_dims=(0,),
                                 start_index_map=(0,)),
      slice_sizes=(1,), mode=lax.GatherScatterMode.PROMISE_IN_BOUNDS)


@functools.partial(
    pl.kernel,
    out_type=(
        jax.ShapeDtypeStruct((NW, NPAD), i32),
        jax.ShapeDtypeStruct((NW, NW), i32),
    ),
    mesh=_mesh,
    compiler_params=_params,
    scratch_types=[
        pltpu.VMEM((NPAD,), i32),
        pltpu.VMEM((2, CH), i32),
        pltpu.VMEM((NW,), i32),
        pltpu.SemaphoreType.DMA,
        pltpu.SemaphoreType.DMA,
    ],
)
def _hist_kernel(src_hbm, hist_hbm, psum_hbm, hist_v, src_v, psum_v, sem0,
                 sem1):
  wid = _wid()

  @plsc.parallel_loop(0, NPAD // L, unroll=8)
  def _(i):
    hist_v[pl.ds(i * L, L)] = jnp.zeros((L,), i32)

  def src_copy(ci, k, sem):
    base = wid * EW + ci * CH
    return pltpu.make_async_copy(src_hbm.at[pl.ds(base, CH)], src_v.at[k],
                                 sem)

  def count_chunk(k):
    @plsc.parallel_loop(0, CH // L, unroll=8)
    def _(i):
      v = src_v[k, pl.ds(i * L, L)]
      cnt, last = plsc.scan_count(v)
      plsc.addupdate_scatter(hist_v, [v], cnt, mask=last)

  src_copy(0, 0, sem0).start()

  def chunk_body(g, _):
    ci = 2 * g
    src_copy(ci + 1, 1, sem1).start()
    src_copy(ci, 0, sem0).wait()
    count_chunk(0)

    @pl.when(ci + 2 < NCH)
    def _():
      src_copy(ci + 2, 0, sem0).start()

    src_copy(ci + 1, 1, sem1).wait()
    count_chunk(1)
    return 0

  lax.fori_loop(0, NCH // 2, chunk_body, 0)

  lane0 = lax.iota(i32, L) == 0
  for r in range(NW):
    def sum_body(j, acc):
      return acc + hist_v[pl.ds(r * NB + j * L, L)]

    acc = lax.fori_loop(0, NB // L, sum_body, jnp.zeros((L,), i32))
    total = jnp.sum(acc)
    plsc.store_scatter(
        psum_v, [jnp.full((L,), r, i32)], jnp.full((L,), total, i32),
        mask=lane0)

  pltpu.sync_copy(hist_v, hist_hbm.at[wid])
  pltpu.sync_copy(psum_v, psum_hbm.at[wid])


@functools.partial(
    pl.kernel,
    out_type=(
        jax.ShapeDtypeStruct((NPAD,), i32),
        jax.ShapeDtypeStruct((NW, NPAD), i32),
    ),
    mesh=_mesh,
    compiler_params=_params,
    scratch_types=[
        pltpu.VMEM((NW, NW), i32),
        pltpu.VMEM((NW, CB), i32),
        pltpu.VMEM((NW, CB), i32),
        pltpu.VMEM((CB,), i32),
    ],
)
def _offsets_kernel(hist_hbm, psum_hbm, splits_hbm, woff_hbm, psum_v, hcol_v,
                    woff_v, spl_v):
  wid = _wid()
  pltpu.sync_copy(psum_hbm, psum_v)

  acc0 = jnp.zeros((L,), i32)
  acc1 = jnp.zeros((L,), i32)
  for w2 in range(NW):
    acc0 = acc0 + psum_v[w2, pl.ds(0, L)]
    acc1 = acc1 + psum_v[w2, pl.ds(L, L)]
  iota = lax.iota(i32, L)
  zero = jnp.zeros((L,), i32)
  base = jnp.sum(jnp.where(iota < wid, acc0, zero)) + jnp.sum(
      jnp.where(iota + L < wid, acc1, zero))

  def sub_chunk(k, carry):
    off = wid * NB + k * CB
    pltpu.sync_copy(hist_hbm.at[:, pl.ds(off, CB)], hcol_v)

    def vec_body(i, c):
      tot = jnp.zeros((L,), i32)
      for w2 in range(NW):
        tot = tot + hcol_v[w2, pl.ds(i * L, L)]
      incl = plsc.cumsum(tot) + jnp.full((L,), c, i32)
      spl_v[pl.ds(i * L, L)] = incl
      run = incl - tot
      for w2 in range(NW):
        woff_v[w2, pl.ds(i * L, L)] = run
        run = run + hcol_v[w2, pl.ds(i * L, L)]
      return c + jnp.sum(tot)

    carry = lax.fori_loop(0, CB // L, vec_body, carry)
    pltpu.sync_copy(woff_v, woff_hbm.at[:, pl.ds(off, CB)])
    pltpu.sync_copy(spl_v, splits_hbm.at[pl.ds(off, CB)])
    return carry

  lax.fori_loop(0, NB // CB, sub_chunk, base)


@functools.partial(
    pl.kernel,
    out_type=(
        jax.ShapeDtypeStruct((TOTUP, L), i32),
        jax.ShapeDtypeStruct((TOTUP, L), i32),
        jax.ShapeDtypeStruct((NW, NRBP + L), i32),
        jax.ShapeDtypeStruct((NW, NRBP + L), i32),
    ),
    mesh=_mesh,
    compiler_params=_params,
    scratch_types=[
        pltpu.VMEM((NPAD,), i32),
        pltpu.VMEM((W,), i32),
        pltpu.VMEM((W,), i32),
        pltpu.VMEM((W,), i32),
        pltpu.VMEM((W,), i32),
        pltpu.VMEM((MAXU, L), i32),
        pltpu.VMEM((MAXU, L), i32),
        pltpu.VMEM((NRBP,), i32),
        pltpu.VMEM((NRBP,), i32),
        pltpu.VMEM((NRBP,), i32),
        pltpu.VMEM((NRBP,), i32),
        pltpu.VMEM((NRBP,), i32),
        pltpu.VMEM((NRBP + L,), i32),
        pltpu.VMEM((MAXU,), i32),
        pltpu.VMEM((MAXU // RCH, RCH), i32),
        pltpu.SemaphoreType.DMA,
        pltpu.SemaphoreType.DMA,
        pltpu.SemaphoreType.DMA,
    ],
)
def _bucketize_kernel(src_hbm, tgt_hbm, woff_hbm, hist_hbm, bpos_hbm,
                      btgt_hbm, sstart_hbm, send_hbm, woff_v, src_v, tgt_v,
                      posw_v, cntw_v, spos_v, stgt_v, h_v, pbu_v, loffw_v,
                      loffs_v, curg_v, sst_v, rid_v, uidx_v, fsem, ssem,
                      tsem):
  wid = _wid()
  zero16 = jnp.zeros((L,), i32)
  iota = lax.iota(i32, L)

  def zero_tab(tab):
    for q in range(NRBP // L):
      tab[pl.ds(q * L, L)] = zero16

  def src_copy(win):
    base = wid * EW + win * W
    return pltpu.make_async_copy(src_hbm.at[pl.ds(base, W)], src_v, ssem)

  def tgt_copy(win):
    base = wid * EW + win * W
    return pltpu.make_async_copy(tgt_hbm.at[pl.ds(base, W)], tgt_v, tsem)

  def window_positions(win):
    zero_tab(h_v)

    def vec_body(i, _):
      v = src_v[pl.ds(i * L, L)]
      cnt, last = plsc.scan_count(v)
      b = plsc.load_gather(woff_v, [v])
      pos = b + cnt - 1
      plsc.store_scatter(woff_v, [v], b + cnt, mask=last)
      posw_v[pl.ds(i * L, L)] = pos
      r = lax.shift_right_logical(pos, 16)
      cnt2, last2 = plsc.scan_count(r)
      plsc.addupdate_scatter(h_v, [r], cnt2, mask=last2)
      cntw_v[pl.ds(i * L, L)] = cnt2 + last2.astype(i32) * 256
      return 0

    lax.fori_loop(0, W // L, vec_body, 0)

  def piece_units(q):
    h = h_v[pl.ds(q * L, L)]
    return jnp.maximum(lax.shift_right_logical(h + 15, 4), 1)

  pltpu.sync_copy(woff_hbm.at[wid], woff_v)
  zero_tab(pbu_v)

  def seg_add(keys, vals, valid):
    vz = jnp.where(valid, vals, jnp.zeros((L,), i32))
    s = plsc.cumsum(vz)
    cnt, last_m = plsc.scan_count(keys, mask=valid)
    prev_i = iota - cnt
    prev_s = _take16(s, jnp.maximum(prev_i, jnp.zeros((L,), i32)))
    prev_s = jnp.where(prev_i < jnp.zeros((L,), i32),
                       jnp.zeros((L,), i32), prev_s)
    plsc.addupdate_scatter(pbu_v, [keys], s - prev_s, mask=last_m)

  HCB = 3_136
  for hc in range(NPAD // HCB):
    pltpu.sync_copy(hist_hbm.at[wid, pl.ds(hc * HCB, HCB)],
                    posw_v.at[pl.ds(0, HCB)])

    def pb_body(i, _):
      wo = woff_v[pl.ds(hc * HCB + i * L, L)]
      c = posw_v[pl.ds(i * L, L)]
      r0 = lax.shift_right_logical(wo, 16)
      cap = jnp.full((L,), BW, i32) - jnp.bitwise_and(
          wo, jnp.full((L,), BW - 1, i32))
      a0 = jnp.minimum(c, cap)
      seg_add(r0, a0, jnp.full((L,), True))

      def spill_cond(st):
        rem, _ = st
        return jnp.any(rem > jnp.zeros((L,), i32))

      def spill_body(st):
        rem, rr = st
        a = jnp.minimum(rem, jnp.full((L,), BW, i32))
        seg_add(rr, a, rem > jnp.zeros((L,), i32))
        return rem - a, rr + jnp.ones((L,), i32)

      lax.while_loop(spill_cond, spill_body,
                     (c - a0, r0 + jnp.ones((L,), i32)))
      return 0

    lax.fori_loop(0, HCB // L, pb_body, 0)

  carry = jnp.zeros((), i32)
  for q in range(NRBP // L):
    pbw = pbu_v[pl.ds(q * L, L)]
    au = lax.shift_right_logical(pbw, 4) + jnp.full((L,), ALLOC_SLACK, i32)
    incl = plsc.cumsum(au) + jnp.full((L,), carry, i32)
    curg_v[pl.ds(q * L, L)] = incl - au + wid * SPU
    sst_v[pl.ds(q * L, L)] = incl - au + wid * SPU
    carry = carry + jnp.sum(au)
  sst_v[pl.ds(NRBP, L)] = jnp.full((L,), carry, i32) + wid * SPU
  pltpu.sync_copy(sst_v, sstart_hbm.at[wid])

  pltpu.sync_copy(woff_hbm.at[wid], woff_v)
  sent16 = jnp.full((L,), -1, i32)
  src_copy(0).start()

  def sweep2_body(win, _):
    tgt_copy(win).start()
    src_copy(win).wait()
    window_positions(win)

    @pl.when(win + 1 < NWIN)
    def _():
      src_copy(win + 1).start()

    c2 = jnp.zeros((), i32)
    for q in range(NRBP // L):
      pu = piece_units(q)
      incl = plsc.cumsum(pu) + jnp.full((L,), c2, i32)
      loffw_v[pl.ds(q * L, L)] = (incl - pu) * L
      loffs_v[pl.ds(q * L, L)] = (incl - pu) * L
      c2 = c2 + jnp.sum(pu)
    total_u = c2

    @pl.when(win > 0)
    def _():
      for j in range(MAXU // RCH):
        pltpu.make_async_copy(spos_v.at[pl.ds(j * RCH, RCH)],
                              bpos_hbm.at[uidx_v.at[j]], fsem).wait()
        pltpu.make_async_copy(stgt_v.at[pl.ds(j * RCH, RCH)],
                              btgt_hbm.at[uidx_v.at[j]], fsem).wait()

    @plsc.parallel_loop(0, MAXU, unroll=8)
    def _(u):
      spos_v[u, :] = sent16

    tgt_copy(win).wait()

    def sort_body(i, _):
      pos = posw_v[pl.ds(i * L, L)]
      tgt = tgt_v[pl.ds(i * L, L)]
      cl = cntw_v[pl.ds(i * L, L)]
      cnt = jnp.bitwise_and(cl, 255)
      last = lax.shift_right_logical(cl, 8) > jnp.zeros((L,), i32)
      r = lax.shift_right_logical(pos, 16)
      bw = plsc.load_gather(loffw_v, [r])
      idx = bw + cnt - 1
      plsc.store_scatter(loffw_v, [r], bw + cnt, mask=last)
      ir = lax.shift_right_logical(idx, 4)
      ic = jnp.bitwise_and(idx, 15)
      plsc.store_scatter(spos_v, [ir, ic], pos)
      plsc.store_scatter(stgt_v, [ir, ic], tgt)
      return 0

    lax.fori_loop(0, W // L, sort_body, 0)

    def rz_body(u, _):
      rid_v[pl.ds(u * L, L)] = zero16
      return 0

    lax.fori_loop(0, MAXU // L, rz_body, 0)
    for q in range(NRBP // L):
      rq = iota + q * L
      starts_u = lax.shift_right_logical(loffs_v[pl.ds(q * L, L)], 4)
      plsc.store_scatter(rid_v, [starts_u], rq,
                         mask=rq < jnp.full((L,), NRB, i32))
    mcarry = jnp.zeros((), i32)
    for u in range(MAXU // L):
      m = plsc.cummax(rid_v[pl.ds(u * L, L)])
      m = jnp.maximum(m, jnp.full((L,), mcarry, i32))
      rid_v[pl.ds(u * L, L)] = m
      mcarry = jnp.max(m)

    for u in range(MAXU // L):
      uu = iota + u * L
      r = rid_v[pl.ds(u * L, L)]
      start_u = lax.shift_right_logical(plsc.load_gather(loffs_v, [r]), 4)
      dst = plsc.load_gather(curg_v, [r]) + uu - start_u
      uidx_v[u // 8, pl.ds((u % 8) * L, L)] = jnp.where(
          uu < jnp.full((L,), total_u, i32), dst,
          jnp.full((L,), TRASH, i32) + uu)

    for q in range(NRBP // L):
      curg_v[pl.ds(q * L, L)] += piece_units(q)

    for j in range(MAXU // RCH):
      pltpu.async_copy(spos_v.at[pl.ds(j * RCH, RCH)],
                       bpos_hbm.at[uidx_v.at[j]], fsem)
      pltpu.async_copy(stgt_v.at[pl.ds(j * RCH, RCH)],
                       btgt_hbm.at[uidx_v.at[j]], fsem)
    return 0

  lax.fori_loop(0, NWIN, sweep2_body, 0)
  for j in range(MAXU // RCH):
    pltpu.make_async_copy(spos_v.at[pl.ds(j * RCH, RCH)],
                          bpos_hbm.at[uidx_v.at[j]], fsem).wait()
    pltpu.make_async_copy(stgt_v.at[pl.ds(j * RCH, RCH)],
                          btgt_hbm.at[uidx_v.at[j]], fsem).wait()
  for q in range(NRBP // L):
    sst_v[pl.ds(q * L, L)] = curg_v[pl.ds(q * L, L)]
  pltpu.sync_copy(sst_v, send_hbm.at[wid])


@functools.partial(
    pl.kernel,
    out_type=jax.ShapeDtypeStruct((E,), i32),
    mesh=_mesh,
    compiler_params=_params,
    scratch_types=[
        pltpu.VMEM((BW,), i32),
        pltpu.VMEM((RCH, L), i32),
        pltpu.VMEM((RCH, L), i32),
        pltpu.VMEM((RCH, L), i32),
        pltpu.VMEM((RCH, L), i32),
        pltpu.VMEM((NW, NRBP + L), i32),
        pltpu.VMEM((NW, NRBP + L), i32),
        pltpu.SemaphoreType.DMA,
        pltpu.SemaphoreType.DMA,
    ],
)
def _place_kernel(bpos_hbm, btgt_hbm, sstart_hbm, send_hbm, out_hbm, stage_v,
                  pos_a, tgt_a, pos_b, tgt_b, sst_v, sse_v, sem_a, sem_b):
  wid = _wid()
  pltpu.sync_copy(sstart_hbm, sst_v)
  pltpu.sync_copy(send_hbm, sse_v)

  def span_of(w2, b):
    su = sst_v[w2, pl.ds(b, L)][0]
    eu = sse_v[w2, pl.ds(b, L)][0]
    return su, eu

  def fire(pos_v, tgt_v, au, sem):
    pltpu.make_async_copy(bpos_hbm.at[pl.ds(au, RCH)], pos_v, sem).start()
    pltpu.make_async_copy(btgt_hbm.at[pl.ds(au, RCH)], tgt_v, sem).start()

  def drain(pos_v, tgt_v, au, sem):
    pltpu.make_async_copy(bpos_hbm.at[pl.ds(au, RCH)], pos_v, sem).wait()
    pltpu.make_async_copy(btgt_hbm.at[pl.ds(au, RCH)], tgt_v, sem).wait()

  def do_bucket(b, flush_words):
    bbase = b * BW

    def process_chunk(pos_v, tgt_v, rem):
      @plsc.parallel_loop(0, rem, unroll=4)
      def _(u):
        pos = pos_v[u, :]
        tgt = tgt_v[u, :]
        ok = pos >= jnp.zeros((L,), i32)
        rel = pos - jnp.full((L,), bbase, i32)
        plsc.store_scatter(stage_v, [rel], tgt, mask=ok)

    def process_span(pos_v, tgt_v, sem, su, eu):
      drain(pos_v, tgt_v, su, sem)
      process_chunk(pos_v, tgt_v, jnp.minimum(eu - su, RCH))

      def chunk_body(cu, _):
        au = su + cu * RCH
        pltpu.sync_copy(bpos_hbm.at[pl.ds(au, RCH)], pos_v)
        pltpu.sync_copy(btgt_hbm.at[pl.ds(au, RCH)], tgt_v)
        process_chunk(pos_v, tgt_v, jnp.minimum(eu - au, RCH))
        return 0

      nchunk = lax.shift_right_logical(eu - su + RCH - 1, 7)
      lax.fori_loop(1, nchunk, chunk_body, 0)

    sua0, _ = span_of(0, b)
    fire(pos_a, tgt_a, sua0, sem_a)

    def pair_body(p, _):
      w2a = 2 * p
      sua, eua = span_of(w2a, b)
      sub, eub = span_of(w2a + 1, b)
      fire(pos_b, tgt_b, sub, sem_b)
      process_span(pos_a, tgt_a, sem_a, sua, eua)

      @pl.when(p + 1 < NW // 2)
      def _():
        sun, _ = span_of(w2a + 2, b)
        fire(pos_a, tgt_a, sun, sem_a)

      process_span(pos_b, tgt_b, sem_b, sub, eub)
      return 0

    lax.fori_loop(0, NW // 2, pair_body, 0)
    pltpu.sync_copy(stage_v.at[pl.ds(0, flush_words)],
                    out_hbm.at[pl.ds(b * BW, flush_words)])

  for j in range(3):
    do_bucket(wid + 32 * j, BW)

  @pl.when(wid == 0)
  def _():
    do_bucket(jnp.full((), 96, i32), BW)

  @pl.when(wid == 1)
  def _():
    do_bucket(jnp.full((), 97, i32), LASTB)


@jax.jit
def _crs_neighbor(edge_index):
  src = edge_index[0].astype(i32)
  tgt = edge_index[1].astype(i32)
  hist, psum = _hist_kernel(src)
  splits_body, woff = _offsets_kernel(hist, psum)
  bpos, btgt, sstart, send = _bucketize_kernel(src, tgt, woff, hist)
  nbr = _place_kernel(bpos, btgt, sstart, send)
  splits = jnp.concatenate(
      [jnp.zeros((1,), i32), splits_body[:N]]).astype(jnp.int64)
  return nbr.astype(jnp.int64), splits


def kernel(edge_index, length):
  del length
  return _crs_neighbor(edge_index)
